# Initial kernel scaffold; baseline (speedup 1.0000x reference)
import jax
import jax.numpy as jnp
from jax import lax
from jax.experimental import pallas as pl
from jax.experimental.pallas import tpu as pltpu
from jax.experimental.pallas import tpu_sc as plsc

N = 10000
E = 160000
D = 256
NH = 16
HD = 16


BN = 400


def _ln_relu(x, g, b):
    mu = jnp.mean(x, axis=-1, keepdims=True)
    var = jnp.mean((x - mu) ** 2, axis=-1, keepdims=True)
    x = (x - mu) * lax.rsqrt(var + 1e-5) * g + b
    return jnp.maximum(x, 0.0)


def _k1_body(h_ref, wq1_ref, bq1_ref, gq_ref, beq_ref, wq2_ref, bq2_ref,
             wk_hi_ref, wk_hj_ref, wv_hi_ref, wv_hj_ref,
             tdst_ref, tsrc_ref):
    h = h_ref[...]
    x = jnp.dot(h, wq1_ref[...], preferred_element_type=jnp.float32) + bq1_ref[...]
    x = _ln_relu(x, gq_ref[...], beq_ref[...])
    q = jnp.dot(x, wq2_ref[...], preferred_element_type=jnp.float32) + bq2_ref[...]
    tdst_ref[:, 0:D] = q
    tdst_ref[:, D:2 * D] = jnp.dot(h, wk_hi_ref[...], preferred_element_type=jnp.float32)
    tdst_ref[:, 2 * D:3 * D] = jnp.dot(h, wv_hi_ref[...], preferred_element_type=jnp.float32)
    tsrc_ref[:, 0:D] = jnp.dot(h, wk_hj_ref[...], preferred_element_type=jnp.float32)
    tsrc_ref[:, D:2 * D] = jnp.dot(h, wv_hj_ref[...], preferred_element_type=jnp.float32)


def _k1(h, wq1, bq1, gq, beq, wq2, bq2, wk_hi, wk_hj, wv_hi, wv_hj):
    nb = N // BN
    full = lambda shape: pl.BlockSpec(shape, lambda i: (0, 0))
    return pl.pallas_call(
        _k1_body,
        grid=(nb,),
        in_specs=[
            pl.BlockSpec((BN, D), lambda i: (i, 0)),
            full((D, D)), full((1, D)), full((1, D)), full((1, D)),
            full((D, D)), full((1, D)),
            full((D, D)), full((D, D)), full((D, D)), full((D, D)),
        ],
        out_specs=[
            pl.BlockSpec((BN, 3 * D), lambda i: (i, 0)),
            pl.BlockSpec((BN, 2 * D), lambda i: (i, 0)),
        ],
        out_shape=[
            jax.ShapeDtypeStruct((N, 3 * D), jnp.float32),
            jax.ShapeDtypeStruct((N, 2 * D), jnp.float32),
        ],
    )(h, wq1, bq1, gq, beq, wq2, bq2, wk_hi, wk_hj, wv_hi, wv_hj)


EPW = E // 32
CH = 40
NCH = EPW // CH


def _k2_body(tdst_hbm, tsrc_hbm, dst_hbm, src_hbm, gd_hbm, gs_hbm,
             idxd_v, idxs_v, rowsd_v, rowss_v, semd, sems):
    wid = lax.axis_index("s") * 2 + lax.axis_index("c")
    base = wid * EPW

    def step(i, carry):
        off = base + i * CH
        pltpu.sync_copy(dst_hbm.at[pl.ds(off, CH)], idxd_v)
        pltpu.sync_copy(src_hbm.at[pl.ds(off, CH)], idxs_v)
        cpd = pltpu.async_copy(tdst_hbm.at[idxd_v], rowsd_v, semd)
        cps = pltpu.async_copy(tsrc_hbm.at[idxs_v], rowss_v, sems)
        cpd.wait()
        cps.wait()
        pltpu.sync_copy(rowsd_v, gd_hbm.at[pl.ds(off, CH)])
        pltpu.sync_copy(rowss_v, gs_hbm.at[pl.ds(off, CH)])
        return carry

    lax.fori_loop(0, NCH, step, 0)


def _k2(tdst, tsrc, dst, src):
    mesh = plsc.VectorSubcoreMesh(core_axis_name="c", subcore_axis_name="s")
    f = pl.kernel(
        _k2_body,
        mesh=mesh,
        out_type=[
            jax.ShapeDtypeStruct((E, 3 * D), jnp.float32),
            jax.ShapeDtypeStruct((E, 2 * D), jnp.float32),
        ],
        scratch_types=[
            pltpu.VMEM((CH,), jnp.int32),
            pltpu.VMEM((CH,), jnp.int32),
            pltpu.VMEM((CH, 3 * D), jnp.float32),
            pltpu.VMEM((CH, 2 * D), jnp.float32),
            pltpu.SemaphoreType.DMA,
            pltpu.SemaphoreType.DMA,
        ],
    )
    return f(tdst, tsrc, dst, src)


BE = 640


def _k3_body(efrf_ref, gd_ref, gs_ref, wer_ref, wew_ref,
             bk1_ref, gk_ref, bek_ref, wk2_ref, bk2_ref,
             bv1_ref, gv_ref, bev_ref, wv2_ref, bv2_ref,
             hmat_ref, ewb_ref,
             s_ref, v2_ref, m_ref):
    efrf = efrf_ref[...]
    pre = jnp.dot(efrf, wer_ref[...], preferred_element_type=jnp.float32)
    pre_k = pre[:, 0:D] + gd_ref[:, D:2 * D] + gs_ref[:, 0:D] + bk1_ref[...]
    pre_v = pre[:, D:2 * D] + gd_ref[:, 2 * D:3 * D] + gs_ref[:, D:2 * D] + bv1_ref[...]
    xk = _ln_relu(pre_k, gk_ref[...], bek_ref[...])
    xv = _ln_relu(pre_v, gv_ref[...], bev_ref[...])
    k = jnp.dot(xk, wk2_ref[...], preferred_element_type=jnp.float32) + bk2_ref[...]
    v = jnp.dot(xv, wv2_ref[...], preferred_element_type=jnp.float32) + bv2_ref[...]
    qd = gd_ref[:, 0:D]
    s = jnp.dot(qd * k, hmat_ref[...], preferred_element_type=jnp.float32) * 0.25
    ew_pre = jnp.dot(efrf, wew_ref[...], preferred_element_type=jnp.float32)
    e_w = jax.nn.sigmoid(ew_pre[:, 0:1] + ewb_ref[0, 0])
    s_ref[...] = s
    v2_ref[...] = v * e_w
    bmax = jnp.max(s)
    i = pl.program_id(0)

    @pl.when(i == 0)
    def _():
        m_ref[0, 0] = bmax

    @pl.when(i > 0)
    def _():
        m_ref[0, 0] = jnp.maximum(m_ref[0, 0], bmax)


def _k3(efrf, gd, gs, wer, wew, bk1, gk, bek, wk2, bk2, bv1, gv, bev, wv2, bv2, hmat, ewb):
    nb = E // BE
    full = lambda shape: pl.BlockSpec(shape, lambda i: (0, 0))
    return pl.pallas_call(
        _k3_body,
        grid=(nb,),
        in_specs=[
            pl.BlockSpec((BE, 80), lambda i: (i, 0)),
            pl.BlockSpec((BE, 3 * D), lambda i: (i, 0)),
            pl.BlockSpec((BE, 2 * D), lambda i: (i, 0)),
            full((80, 2 * D)), full((80, 8)),
            full((1, D)), full((1, D)), full((1, D)), full((D, D)), full((1, D)),
            full((1, D)), full((1, D)), full((1, D)), full((D, NH)), full((1, NH)),
            full((D, NH)), full((1, 1)),
        ],
        out_specs=[
            pl.BlockSpec((BE, NH), lambda i: (i, 0)),
            pl.BlockSpec((BE, NH), lambda i: (i, 0)),
            pl.BlockSpec((1, 1), lambda i: (0, 0)),
        ],
        out_shape=[
            jax.ShapeDtypeStruct((E, NH), jnp.float32),
            jax.ShapeDtypeStruct((E, NH), jnp.float32),
            jax.ShapeDtypeStruct((1, 1), jnp.float32),
        ],
    )(efrf, gd, gs, wer, wew, bk1, gk, bek, wk2, bk2, bv1, gv, bev, wv2, bv2, hmat, ewb)


def _k4_body(s_ref, v2_ref, rx_ref, m_ref, u_ref):
    t = jnp.exp(s_ref[...] - m_ref[0, 0])
    p = t * v2_ref[...]
    rx = rx_ref[...]
    u_ref[:, 0:NH] = t
    u_ref[:, NH:2 * NH] = p * rx[:, 0:1]
    u_ref[:, 2 * NH:3 * NH] = p * rx[:, 1:2]
    u_ref[:, 3 * NH:4 * NH] = p * rx[:, 2:3]


def _k4(s, v2, rel_x, m):
    nb = E // BE
    return pl.pallas_call(
        _k4_body,
        grid=(nb,),
        in_specs=[
            pl.BlockSpec((BE, NH), lambda i: (i, 0)),
            pl.BlockSpec((BE, NH), lambda i: (i, 0)),
            pl.BlockSpec((BE, 3), lambda i: (i, 0)),
            pl.BlockSpec((1, 1), lambda i: (0, 0)),
        ],
        out_specs=pl.BlockSpec((BE, 4 * NH), lambda i: (i, 0)),
        out_shape=jax.ShapeDtypeStruct((E, 4 * NH), jnp.float32),
    )(s, v2, rel_x, m)


def _k5_body(u_hbm, dst_hbm, zeros_hbm, s_hbm, idx_v, u_v, shared):
    cid = lax.axis_index("c")
    sid = lax.axis_index("s")
    base = (cid * 16 + sid) * EPW

    @pl.when(sid == 0)
    def _():
        pltpu.sync_copy(zeros_hbm, shared)

    plsc.subcore_barrier()

    def step(i, carry):
        off = base + i * CH
        pltpu.sync_copy(dst_hbm.at[pl.ds(off, CH)], idx_v)
        pltpu.sync_copy(u_hbm.at[pl.ds(off, CH)], u_v)
        pltpu.sync_copy(u_v, shared.at[idx_v], add=True)
        return carry

    lax.fori_loop(0, NCH, step, 0)
    plsc.subcore_barrier()

    @pl.when(sid == 0)
    def _():
        pltpu.sync_copy(shared, s_hbm.at[cid])


def _k5(u, dst, zeros):
    mesh = plsc.VectorSubcoreMesh(core_axis_name="c", subcore_axis_name="s")
    f = pl.kernel(
        _k5_body,
        mesh=mesh,
        out_type=jax.ShapeDtypeStruct((2, N, 4 * NH), jnp.float32),
        scratch_types=[
            pltpu.VMEM((CH,), jnp.int32),
            pltpu.VMEM((CH, 4 * NH), jnp.float32),
            pltpu.VMEM_SHARED((N, 4 * NH), jnp.float32),
        ],
    )
    return f(u, dst, zeros)


BN6 = 2000


def _k6_body(s_ref, o_ref):
    den = s_ref[0, :, 0:NH] + s_ref[1, :, 0:NH]
    r = 1.0 / (den + 1e-16)
    for c in range(3):
        a = s_ref[0, :, (c + 1) * NH:(c + 2) * NH] + s_ref[1, :, (c + 1) * NH:(c + 2) * NH]
        o_ref[:, c:c + 1] = jnp.sum(a * r, axis=-1, keepdims=True) * (1.0 / NH)
    o_ref[:, 3:4] = jnp.zeros((BN6, 1), jnp.float32)


def _k6(s):
    nb = N // BN6
    return pl.pallas_call(
        _k6_body,
        grid=(nb,),
        in_specs=[pl.BlockSpec((2, BN6, 4 * NH), lambda i: (0, i, 0))],
        out_specs=pl.BlockSpec((BN6, 4), lambda i: (i, 0)),
        out_shape=jax.ShapeDtypeStruct((N, 4), jnp.float32),
    )(s)


def kernel(h, rel_x, r_feat, edge_feat, edge_index, xk_W1, xk_b1, xk_g, xk_be,
           xk_W2, xk_b2, xv_W1, xv_b1, xv_g, xv_be, xv_W2, xv_b2,
           xq_W1, xq_b1, xq_g, xq_be, xq_W2, xq_b2, ew_W, ew_b):
    src = edge_index[0]
    dst = edge_index[1]
    row = lambda x: x.reshape(1, -1)

    tdst, tsrc = _k1(
        h, xq_W1, row(xq_b1), row(xq_g), row(xq_be), xq_W2, row(xq_b2),
        xk_W1[80:336], xk_W1[336:592], xv_W1[80:336], xv_W1[336:592])

    gd, gs = _k2(tdst, tsrc, dst, src)

    efrf = jnp.concatenate([edge_feat, r_feat], axis=1)
    wer = jnp.concatenate([xk_W1[0:80], xv_W1[0:80]], axis=1)
    wew = jnp.zeros((80, 8), jnp.float32).at[16:80, 0].set(ew_W[:, 0])
    hmat = (jax.lax.broadcasted_iota(jnp.int32, (D, NH), 0) // HD
            == jax.lax.broadcasted_iota(jnp.int32, (D, NH), 1)).astype(jnp.float32)

    s, v2, m = _k3(efrf, gd, gs, wer, wew,
                   row(xk_b1), row(xk_g), row(xk_be), xk_W2, row(xk_b2),
                   row(xv_b1), row(xv_g), row(xv_be), xv_W2, row(xv_b2),
                   hmat, ew_b.reshape(1, 1))

    u = _k4(s, v2, rel_x, m)

    sacc = _k5(u, dst, jnp.zeros((N, 4 * NH), jnp.float32))

    out = _k6(sacc)
    return out[:, :3]


# speedup vs baseline: 12.3653x; 12.3653x over previous
import jax
import jax.numpy as jnp
from jax import lax
from jax.experimental import pallas as pl
from jax.experimental.pallas import tpu as pltpu
from jax.experimental.pallas import tpu_sc as plsc

N = 10000
E = 160000
D = 256
NH = 16
HD = 16
UW = 128


BN = 400


def _ln_relu(x, g, b):
    mu = jnp.mean(x, axis=-1, keepdims=True)
    var = jnp.mean((x - mu) ** 2, axis=-1, keepdims=True)
    x = (x - mu) * lax.rsqrt(var + 1e-5) * g + b
    return jnp.maximum(x, 0.0)


def _k1_body(h_ref, wq1_ref, bq1_ref, gq_ref, beq_ref, wq2_ref, bq2_ref,
             wk_hi_ref, wk_hj_ref, wv_hi_ref, wv_hj_ref,
             tdst_ref, tsrc_ref):
    h = h_ref[...]
    x = jnp.dot(h, wq1_ref[...], preferred_element_type=jnp.float32) + bq1_ref[...]
    x = _ln_relu(x, gq_ref[...], beq_ref[...])
    q = jnp.dot(x, wq2_ref[...], preferred_element_type=jnp.float32) + bq2_ref[...]
    tdst_ref[:, 0:D] = q
    tdst_ref[:, D:2 * D] = jnp.dot(h, wk_hi_ref[...], preferred_element_type=jnp.float32)
    tdst_ref[:, 2 * D:3 * D] = jnp.dot(h, wv_hi_ref[...], preferred_element_type=jnp.float32)
    tsrc_ref[:, 0:D] = jnp.dot(h, wk_hj_ref[...], preferred_element_type=jnp.float32)
    tsrc_ref[:, D:2 * D] = jnp.dot(h, wv_hj_ref[...], preferred_element_type=jnp.float32)


def _k1(h, wq1, bq1, gq, beq, wq2, bq2, wk_hi, wk_hj, wv_hi, wv_hj):
    nb = N // BN
    full = lambda shape: pl.BlockSpec(shape, lambda i: (0, 0))
    return pl.pallas_call(
        _k1_body,
        grid=(nb,),
        in_specs=[
            pl.BlockSpec((BN, D), lambda i: (i, 0)),
            full((D, D)), full((1, D)), full((1, D)), full((1, D)),
            full((D, D)), full((1, D)),
            full((D, D)), full((D, D)), full((D, D)), full((D, D)),
        ],
        out_specs=[
            pl.BlockSpec((BN, 3 * D), lambda i: (i, 0)),
            pl.BlockSpec((BN, 2 * D), lambda i: (i, 0)),
        ],
        out_shape=[
            jax.ShapeDtypeStruct((N, 3 * D), jnp.float32),
            jax.ShapeDtypeStruct((N, 2 * D), jnp.float32),
        ],
    )(h, wq1, bq1, gq, beq, wq2, bq2, wk_hi, wk_hj, wv_hi, wv_hj)


EPW = E // 32
CH = 40
NCH = EPW // CH


def _k2_body(tdst_hbm, tsrc_hbm, dst_hbm, src_hbm, gd_hbm, gs_hbm,
             idxd_v, idxs_v, rowsd_v, rowss_v, semd, sems):
    wid = lax.axis_index("s") * 2 + lax.axis_index("c")
    base = wid * EPW

    def step(i, carry):
        off = base + i * CH
        pltpu.sync_copy(dst_hbm.at[pl.ds(off, CH)], idxd_v)
        pltpu.sync_copy(src_hbm.at[pl.ds(off, CH)], idxs_v)
        cpd = pltpu.async_copy(tdst_hbm.at[idxd_v], rowsd_v, semd)
        cps = pltpu.async_copy(tsrc_hbm.at[idxs_v], rowss_v, sems)
        cpd.wait()
        cps.wait()
        pltpu.sync_copy(rowsd_v, gd_hbm.at[pl.ds(off, CH)])
        pltpu.sync_copy(rowss_v, gs_hbm.at[pl.ds(off, CH)])
        return carry

    lax.fori_loop(0, NCH, step, 0)


def _k2(tdst, tsrc, dst, src):
    mesh = plsc.VectorSubcoreMesh(core_axis_name="c", subcore_axis_name="s")
    f = pl.kernel(
        _k2_body,
        mesh=mesh,
        out_type=[
            jax.ShapeDtypeStruct((E, 3 * D), jnp.float32),
            jax.ShapeDtypeStruct((E, 2 * D), jnp.float32),
        ],
        scratch_types=[
            pltpu.VMEM((CH,), jnp.int32),
            pltpu.VMEM((CH,), jnp.int32),
            pltpu.VMEM((CH, 3 * D), jnp.float32),
            pltpu.VMEM((CH, 2 * D), jnp.float32),
            pltpu.SemaphoreType.DMA,
            pltpu.SemaphoreType.DMA,
        ],
    )
    return f(tdst, tsrc, dst, src)


BE = 640


def _k3_body(efrf_ref, gd_ref, gs_ref, wer_ref, wew_ref,
             bk1_ref, gk_ref, bek_ref, wk2_ref, bk2_ref,
             bv1_ref, gv_ref, bev_ref, wv2_ref, bv2_ref,
             hmat_ref, ewb_ref,
             s_ref, v2_ref, m_ref):
    efrf = efrf_ref[...]
    pre = jnp.dot(efrf, wer_ref[...], preferred_element_type=jnp.float32)
    pre_k = pre[:, 0:D] + gd_ref[:, D:2 * D] + gs_ref[:, 0:D] + bk1_ref[...]
    pre_v = pre[:, D:2 * D] + gd_ref[:, 2 * D:3 * D] + gs_ref[:, D:2 * D] + bv1_ref[...]
    xk = _ln_relu(pre_k, gk_ref[...], bek_ref[...])
    xv = _ln_relu(pre_v, gv_ref[...], bev_ref[...])
    k = jnp.dot(xk, wk2_ref[...], preferred_element_type=jnp.float32) + bk2_ref[...]
    v = jnp.dot(xv, wv2_ref[...], preferred_element_type=jnp.float32) + bv2_ref[...]
    qd = gd_ref[:, 0:D]
    s = jnp.dot(qd * k, hmat_ref[...], preferred_element_type=jnp.float32) * 0.25
    ew_pre = jnp.dot(efrf, wew_ref[...], preferred_element_type=jnp.float32)
    e_w = jax.nn.sigmoid(ew_pre[:, 0:1] + ewb_ref[0, 0])
    s_ref[...] = s
    v2_ref[...] = v * e_w
    bmax = jnp.max(s)
    i = pl.program_id(0)

    @pl.when(i == 0)
    def _():
        m_ref[0, 0] = bmax

    @pl.when(i > 0)
    def _():
        m_ref[0, 0] = jnp.maximum(m_ref[0, 0], bmax)


def _k3(efrf, gd, gs, wer, wew, bk1, gk, bek, wk2, bk2, bv1, gv, bev, wv2, bv2, hmat, ewb):
    nb = E // BE
    full = lambda shape: pl.BlockSpec(shape, lambda i: (0, 0))
    return pl.pallas_call(
        _k3_body,
        grid=(nb,),
        in_specs=[
            pl.BlockSpec((BE, 80), lambda i: (i, 0)),
            pl.BlockSpec((BE, 3 * D), lambda i: (i, 0)),
            pl.BlockSpec((BE, 2 * D), lambda i: (i, 0)),
            full((80, 2 * D)), full((80, 8)),
            full((1, D)), full((1, D)), full((1, D)), full((D, D)), full((1, D)),
            full((1, D)), full((1, D)), full((1, D)), full((D, NH)), full((1, NH)),
            full((D, NH)), full((1, 1)),
        ],
        out_specs=[
            pl.BlockSpec((BE, NH), lambda i: (i, 0)),
            pl.BlockSpec((BE, NH), lambda i: (i, 0)),
            pl.BlockSpec((1, 1), lambda i: (0, 0), memory_space=pltpu.SMEM),
        ],
        out_shape=[
            jax.ShapeDtypeStruct((E, NH), jnp.float32),
            jax.ShapeDtypeStruct((E, NH), jnp.float32),
            jax.ShapeDtypeStruct((1, 1), jnp.float32),
        ],
    )(efrf, gd, gs, wer, wew, bk1, gk, bek, wk2, bk2, bv1, gv, bev, wv2, bv2, hmat, ewb)


def _k4_body(s_ref, v2_ref, rx_ref, m_ref, u_ref):
    t = jnp.exp(s_ref[...] - m_ref[0, 0])
    p = t * v2_ref[...]
    rx = rx_ref[...]
    u_ref[:, 0:NH] = t
    u_ref[:, NH:2 * NH] = p * rx[:, 0:1]
    u_ref[:, 2 * NH:3 * NH] = p * rx[:, 1:2]
    u_ref[:, 3 * NH:4 * NH] = p * rx[:, 2:3]
    u_ref[:, 4 * NH:UW] = jnp.zeros((BE, UW - 4 * NH), jnp.float32)


def _k4(s, v2, rel_x, m):
    nb = E // BE
    return pl.pallas_call(
        _k4_body,
        grid=(nb,),
        in_specs=[
            pl.BlockSpec((BE, NH), lambda i: (i, 0)),
            pl.BlockSpec((BE, NH), lambda i: (i, 0)),
            pl.BlockSpec((BE, 3), lambda i: (i, 0)),
            pl.BlockSpec((1, 1), lambda i: (0, 0), memory_space=pltpu.SMEM),
        ],
        out_specs=pl.BlockSpec((BE, UW), lambda i: (i, 0)),
        out_shape=jax.ShapeDtypeStruct((E, UW), jnp.float32),
    )(s, v2, rel_x, m)


def _k5_body(u_hbm, dst_hbm, zeros_hbm, s_hbm, idx_v, u_v, shared):
    cid = lax.axis_index("c")
    sid = lax.axis_index("s")
    base = (cid * 16 + sid) * EPW

    @pl.when(sid == 0)
    def _():
        pltpu.sync_copy(zeros_hbm, shared)

    plsc.subcore_barrier()

    def step(i, carry):
        off = base + i * CH
        pltpu.sync_copy(dst_hbm.at[pl.ds(off, CH)], idx_v)
        pltpu.sync_copy(u_hbm.at[pl.ds(off, CH)], u_v)
        pltpu.sync_copy(u_v, shared.at[idx_v], add=True)
        return carry

    lax.fori_loop(0, NCH, step, 0)
    plsc.subcore_barrier()

    @pl.when(sid == 0)
    def _():
        pltpu.sync_copy(shared, s_hbm.at[cid])


def _k5(u, dst, zeros):
    mesh = plsc.VectorSubcoreMesh(core_axis_name="c", subcore_axis_name="s")
    f = pl.kernel(
        _k5_body,
        mesh=mesh,
        out_type=jax.ShapeDtypeStruct((2, N, UW), jnp.float32),
        scratch_types=[
            pltpu.VMEM((CH,), jnp.int32),
            pltpu.VMEM((CH, UW), jnp.float32),
            pltpu.VMEM_SHARED((N, UW), jnp.float32),
        ],
    )
    return f(u, dst, zeros)


BN6 = 2000


def _k6_body(s_ref, o_ref):
    den = s_ref[0, :, 0:NH] + s_ref[1, :, 0:NH]
    r = 1.0 / (den + 1e-16)
    for c in range(3):
        a = s_ref[0, :, (c + 1) * NH:(c + 2) * NH] + s_ref[1, :, (c + 1) * NH:(c + 2) * NH]
        o_ref[:, c:c + 1] = jnp.sum(a * r, axis=-1, keepdims=True) * (1.0 / NH)
    o_ref[:, 3:4] = jnp.zeros((BN6, 1), jnp.float32)


def _k6(s):
    nb = N // BN6
    return pl.pallas_call(
        _k6_body,
        grid=(nb,),
        in_specs=[pl.BlockSpec((2, BN6, UW), lambda i: (0, i, 0))],
        out_specs=pl.BlockSpec((BN6, 4), lambda i: (i, 0)),
        out_shape=jax.ShapeDtypeStruct((N, 4), jnp.float32),
    )(s)


def kernel(h, rel_x, r_feat, edge_feat, edge_index, xk_W1, xk_b1, xk_g, xk_be,
           xk_W2, xk_b2, xv_W1, xv_b1, xv_g, xv_be, xv_W2, xv_b2,
           xq_W1, xq_b1, xq_g, xq_be, xq_W2, xq_b2, ew_W, ew_b):
    src = edge_index[0]
    dst = edge_index[1]
    row = lambda x: x.reshape(1, -1)

    tdst, tsrc = _k1(
        h, xq_W1, row(xq_b1), row(xq_g), row(xq_be), xq_W2, row(xq_b2),
        xk_W1[80:336], xk_W1[336:592], xv_W1[80:336], xv_W1[336:592])

    gd, gs = _k2(tdst, tsrc, dst, src)

    efrf = jnp.concatenate([edge_feat, r_feat], axis=1)
    wer = jnp.concatenate([xk_W1[0:80], xv_W1[0:80]], axis=1)
    wew = jnp.zeros((80, 8), jnp.float32).at[16:80, 0].set(ew_W[:, 0])
    hmat = (jax.lax.broadcasted_iota(jnp.int32, (D, NH), 0) // HD
            == jax.lax.broadcasted_iota(jnp.int32, (D, NH), 1)).astype(jnp.float32)

    s, v2, m = _k3(efrf, gd, gs, wer, wew,
                   row(xk_b1), row(xk_g), row(xk_be), xk_W2, row(xk_b2),
                   row(xv_b1), row(xv_g), row(xv_be), xv_W2, row(xv_b2),
                   hmat, ew_b.reshape(1, 1))

    u = _k4(s, v2, rel_x, m)

    sacc = _k5(u, dst, jnp.zeros((N, UW), jnp.float32))

    out = _k6(sacc)
    return out[:, :3]


# speedup vs baseline: 14.6809x; 1.1873x over previous
import jax
import jax.numpy as jnp
from jax import lax
from jax.experimental import pallas as pl
from jax.experimental.pallas import tpu as pltpu
from jax.experimental.pallas import tpu_sc as plsc

N = 10000
E = 160000
D = 256
NH = 16
HD = 16
UW = 128


BN = 400


def _ln_relu(x, g, b):
    mu = jnp.mean(x, axis=-1, keepdims=True)
    var = jnp.mean((x - mu) ** 2, axis=-1, keepdims=True)
    x = (x - mu) * lax.rsqrt(var + 1e-5) * g + b
    return jnp.maximum(x, 0.0)


def _k1_body(h_ref, wq1_ref, bq1_ref, gq_ref, beq_ref, wq2_ref, bq2_ref,
             wk_hi_ref, wk_hj_ref, wv_hi_ref, wv_hj_ref,
             tdst_ref, tsrc_ref):
    h = h_ref[...]
    x = jnp.dot(h, wq1_ref[...], preferred_element_type=jnp.float32) + bq1_ref[...]
    x = _ln_relu(x, gq_ref[...], beq_ref[...])
    q = jnp.dot(x, wq2_ref[...], preferred_element_type=jnp.float32) + bq2_ref[...]
    tdst_ref[:, 0:D] = q
    tdst_ref[:, D:2 * D] = jnp.dot(h, wk_hi_ref[...], preferred_element_type=jnp.float32)
    tdst_ref[:, 2 * D:3 * D] = jnp.dot(h, wv_hi_ref[...], preferred_element_type=jnp.float32)
    tsrc_ref[:, 0:D] = jnp.dot(h, wk_hj_ref[...], preferred_element_type=jnp.float32)
    tsrc_ref[:, D:2 * D] = jnp.dot(h, wv_hj_ref[...], preferred_element_type=jnp.float32)


def _k1(h, wq1, bq1, gq, beq, wq2, bq2, wk_hi, wk_hj, wv_hi, wv_hj):
    nb = N // BN
    full = lambda shape: pl.BlockSpec(shape, lambda i: (0, 0))
    return pl.pallas_call(
        _k1_body,
        grid=(nb,),
        in_specs=[
            pl.BlockSpec((BN, D), lambda i: (i, 0)),
            full((D, D)), full((1, D)), full((1, D)), full((1, D)),
            full((D, D)), full((1, D)),
            full((D, D)), full((D, D)), full((D, D)), full((D, D)),
        ],
        out_specs=[
            pl.BlockSpec((BN, 3 * D), lambda i: (i, 0)),
            pl.BlockSpec((BN, 2 * D), lambda i: (i, 0)),
        ],
        out_shape=[
            jax.ShapeDtypeStruct((N, 3 * D), jnp.float32),
            jax.ShapeDtypeStruct((N, 2 * D), jnp.float32),
        ],
    )(h, wq1, bq1, gq, beq, wq2, bq2, wk_hi, wk_hj, wv_hi, wv_hj)


EPW = E // 32
CH = 40
NCH = EPW // CH


def _k2_body(tdst_hbm, tsrc_hbm, dst_hbm, src_hbm, gd_hbm, gs_hbm,
             idxd_v, idxs_v, rowsd_v, rowss_v, semd, sems):
    wid = lax.axis_index("s") * 2 + lax.axis_index("c")
    base = wid * EPW

    def step(i, carry):
        off = base + i * CH
        pltpu.sync_copy(dst_hbm.at[pl.ds(off, CH)], idxd_v)
        pltpu.sync_copy(src_hbm.at[pl.ds(off, CH)], idxs_v)
        cpd = pltpu.async_copy(tdst_hbm.at[idxd_v], rowsd_v, semd)
        cps = pltpu.async_copy(tsrc_hbm.at[idxs_v], rowss_v, sems)
        cpd.wait()
        cps.wait()
        pltpu.sync_copy(rowsd_v, gd_hbm.at[pl.ds(off, CH)])
        pltpu.sync_copy(rowss_v, gs_hbm.at[pl.ds(off, CH)])
        return carry

    lax.fori_loop(0, NCH, step, 0)


def _k2(tdst, tsrc, dst, src):
    mesh = plsc.VectorSubcoreMesh(core_axis_name="c", subcore_axis_name="s")
    f = pl.kernel(
        _k2_body,
        mesh=mesh,
        out_type=[
            jax.ShapeDtypeStruct((E, 384), jnp.int32),
            jax.ShapeDtypeStruct((E, 256), jnp.int32),
        ],
        scratch_types=[
            pltpu.VMEM((CH,), jnp.int32),
            pltpu.VMEM((CH,), jnp.int32),
            pltpu.VMEM((CH, 384), jnp.int32),
            pltpu.VMEM((CH, 256), jnp.int32),
            pltpu.SemaphoreType.DMA,
            pltpu.SemaphoreType.DMA,
        ],
    )
    return f(tdst, tsrc, dst, src)


BE = 640


def _unpack(xi32):
    lo = lax.bitcast_convert_type(xi32 << 16, jnp.float32)
    hi = lax.bitcast_convert_type(xi32 & jnp.int32(-65536), jnp.float32)
    return jnp.concatenate([lo, hi], axis=1)


def _k3_body(efrf_ref, gd_ref, gs_ref, wer_ref, wew_ref,
             bk1_ref, gk_ref, bek_ref, wk2_ref, bk2_ref,
             bv1_ref, gv_ref, bev_ref, wv2_ref, bv2_ref,
             hmat_ref, ewb_ref,
             s_ref, v2_ref, m_ref):
    efrf = efrf_ref[...]
    pk_d = _unpack(gd_ref[:, 128:256])
    pv_d = _unpack(gd_ref[:, 256:384])
    pk_s = _unpack(gs_ref[:, 0:128])
    pv_s = _unpack(gs_ref[:, 128:256])
    pre = jnp.dot(efrf, wer_ref[...], preferred_element_type=jnp.float32)
    pre_k = pre[:, 0:D] + pk_d + pk_s + bk1_ref[...]
    pre_v = pre[:, D:2 * D] + pv_d + pv_s + bv1_ref[...]
    xk = _ln_relu(pre_k, gk_ref[...], bek_ref[...])
    xv = _ln_relu(pre_v, gv_ref[...], bev_ref[...])
    k = jnp.dot(xk, wk2_ref[...], preferred_element_type=jnp.float32) + bk2_ref[...]
    v = jnp.dot(xv, wv2_ref[...], preferred_element_type=jnp.float32) + bv2_ref[...]
    qd = _unpack(gd_ref[:, 0:128])
    s = jnp.dot(qd * k, hmat_ref[...], preferred_element_type=jnp.float32) * 0.25
    ew_pre = jnp.dot(efrf, wew_ref[...], preferred_element_type=jnp.float32)
    e_w = jax.nn.sigmoid(ew_pre[:, 0:1] + ewb_ref[0, 0])
    s_ref[...] = s
    v2_ref[...] = v * e_w
    bmax = jnp.max(s)
    i = pl.program_id(0)

    @pl.when(i == 0)
    def _():
        m_ref[0, 0] = bmax

    @pl.when(i > 0)
    def _():
        m_ref[0, 0] = jnp.maximum(m_ref[0, 0], bmax)


def _k3(efrf, gd, gs, wer, wew, bk1, gk, bek, wk2, bk2, bv1, gv, bev, wv2, bv2, hmat, ewb):
    nb = E // BE
    full = lambda shape: pl.BlockSpec(shape, lambda i: (0, 0))
    return pl.pallas_call(
        _k3_body,
        grid=(nb,),
        in_specs=[
            pl.BlockSpec((BE, 80), lambda i: (i, 0)),
            pl.BlockSpec((BE, 384), lambda i: (i, 0)),
            pl.BlockSpec((BE, 256), lambda i: (i, 0)),
            full((80, 2 * D)), full((80, 8)),
            full((1, D)), full((1, D)), full((1, D)), full((D, D)), full((1, D)),
            full((1, D)), full((1, D)), full((1, D)), full((D, NH)), full((1, NH)),
            full((D, NH)), full((1, 1)),
        ],
        out_specs=[
            pl.BlockSpec((BE, NH), lambda i: (i, 0)),
            pl.BlockSpec((BE, NH), lambda i: (i, 0)),
            pl.BlockSpec((1, 1), lambda i: (0, 0), memory_space=pltpu.SMEM),
        ],
        out_shape=[
            jax.ShapeDtypeStruct((E, NH), jnp.float32),
            jax.ShapeDtypeStruct((E, NH), jnp.float32),
            jax.ShapeDtypeStruct((1, 1), jnp.float32),
        ],
    )(efrf, gd, gs, wer, wew, bk1, gk, bek, wk2, bk2, bv1, gv, bev, wv2, bv2, hmat, ewb)


def _k4_body(s_ref, v2_ref, rx_ref, m_ref, u_ref):
    t = jnp.exp(s_ref[...] - m_ref[0, 0])
    p = t * v2_ref[...]
    rx = rx_ref[...]
    u_ref[:, 0:NH] = t
    u_ref[:, NH:2 * NH] = p * rx[:, 0:1]
    u_ref[:, 2 * NH:3 * NH] = p * rx[:, 1:2]
    u_ref[:, 3 * NH:4 * NH] = p * rx[:, 2:3]
    u_ref[:, 4 * NH:UW] = jnp.zeros((BE, UW - 4 * NH), jnp.float32)


def _k4(s, v2, rel_x, m):
    nb = E // BE
    return pl.pallas_call(
        _k4_body,
        grid=(nb,),
        in_specs=[
            pl.BlockSpec((BE, NH), lambda i: (i, 0)),
            pl.BlockSpec((BE, NH), lambda i: (i, 0)),
            pl.BlockSpec((BE, 3), lambda i: (i, 0)),
            pl.BlockSpec((1, 1), lambda i: (0, 0), memory_space=pltpu.SMEM),
        ],
        out_specs=pl.BlockSpec((BE, UW), lambda i: (i, 0)),
        out_shape=jax.ShapeDtypeStruct((E, UW), jnp.float32),
    )(s, v2, rel_x, m)


def _k5_body(u_hbm, dst_hbm, zeros_hbm, s_hbm, idx_v, u_v, shared):
    cid = lax.axis_index("c")
    sid = lax.axis_index("s")
    base = (cid * 16 + sid) * EPW

    @pl.when(sid == 0)
    def _():
        pltpu.sync_copy(zeros_hbm, shared)

    plsc.subcore_barrier()

    def step(i, carry):
        off = base + i * CH
        pltpu.sync_copy(dst_hbm.at[pl.ds(off, CH)], idx_v)
        pltpu.sync_copy(u_hbm.at[pl.ds(off, CH)], u_v)
        pltpu.sync_copy(u_v, shared.at[idx_v], add=True)
        return carry

    lax.fori_loop(0, NCH, step, 0)
    plsc.subcore_barrier()

    @pl.when(sid == 0)
    def _():
        pltpu.sync_copy(shared, s_hbm.at[cid])


def _k5(u, dst, zeros):
    mesh = plsc.VectorSubcoreMesh(core_axis_name="c", subcore_axis_name="s")
    f = pl.kernel(
        _k5_body,
        mesh=mesh,
        out_type=jax.ShapeDtypeStruct((2, N, UW), jnp.float32),
        scratch_types=[
            pltpu.VMEM((CH,), jnp.int32),
            pltpu.VMEM((CH, UW), jnp.float32),
            pltpu.VMEM_SHARED((N, UW), jnp.float32),
        ],
    )
    return f(u, dst, zeros)


BN6 = 2000


def _k6_body(s_ref, o_ref):
    den = s_ref[0, :, 0:NH] + s_ref[1, :, 0:NH]
    r = 1.0 / (den + 1e-16)
    for c in range(3):
        a = s_ref[0, :, (c + 1) * NH:(c + 2) * NH] + s_ref[1, :, (c + 1) * NH:(c + 2) * NH]
        o_ref[:, c:c + 1] = jnp.sum(a * r, axis=-1, keepdims=True) * (1.0 / NH)
    o_ref[:, 3:4] = jnp.zeros((BN6, 1), jnp.float32)


def _k6(s):
    nb = N // BN6
    return pl.pallas_call(
        _k6_body,
        grid=(nb,),
        in_specs=[pl.BlockSpec((2, BN6, UW), lambda i: (0, i, 0))],
        out_specs=pl.BlockSpec((BN6, 4), lambda i: (i, 0)),
        out_shape=jax.ShapeDtypeStruct((N, 4), jnp.float32),
    )(s)


def kernel(h, rel_x, r_feat, edge_feat, edge_index, xk_W1, xk_b1, xk_g, xk_be,
           xk_W2, xk_b2, xv_W1, xv_b1, xv_g, xv_be, xv_W2, xv_b2,
           xq_W1, xq_b1, xq_g, xq_be, xq_W2, xq_b2, ew_W, ew_b):
    src = edge_index[0]
    dst = edge_index[1]
    row = lambda x: x.reshape(1, -1)

    tdst, tsrc = _k1(
        h, xq_W1, row(xq_b1), row(xq_g), row(xq_be), xq_W2, row(xq_b2),
        xk_W1[80:336], xk_W1[336:592], xv_W1[80:336], xv_W1[336:592])

    def pack_block(b):
        b16 = lax.bitcast_convert_type(b.astype(jnp.bfloat16), jnp.uint16)
        lo = b16[:, :128].astype(jnp.uint32)
        hi = b16[:, 128:].astype(jnp.uint32)
        return lax.bitcast_convert_type(lo | (hi << 16), jnp.int32)

    tdst_p = jnp.concatenate([pack_block(tdst[:, i * D:(i + 1) * D]) for i in range(3)], 1)
    tsrc_p = jnp.concatenate([pack_block(tsrc[:, i * D:(i + 1) * D]) for i in range(2)], 1)
    gd, gs = _k2(tdst_p, tsrc_p, dst, src)

    efrf = jnp.concatenate([edge_feat, r_feat], axis=1)
    wer = jnp.concatenate([xk_W1[0:80], xv_W1[0:80]], axis=1)
    wew = jnp.zeros((80, 8), jnp.float32).at[16:80, 0].set(ew_W[:, 0])
    hmat = (jax.lax.broadcasted_iota(jnp.int32, (D, NH), 0) // HD
            == jax.lax.broadcasted_iota(jnp.int32, (D, NH), 1)).astype(jnp.float32)

    s, v2, m = _k3(efrf, gd, gs, wer, wew,
                   row(xk_b1), row(xk_g), row(xk_be), xk_W2, row(xk_b2),
                   row(xv_b1), row(xv_g), row(xv_be), xv_W2, row(xv_b2),
                   hmat, ew_b.reshape(1, 1))

    u = _k4(s, v2, rel_x, m)

    sacc = _k5(u, dst, jnp.zeros((N, UW), jnp.float32))

    out = _k6(sacc)
    return out[:, :3]


# speedup vs baseline: 15.9833x; 1.0887x over previous
import jax
import jax.numpy as jnp
from jax import lax
from jax.experimental import pallas as pl
from jax.experimental.pallas import tpu as pltpu
from jax.experimental.pallas import tpu_sc as plsc

N = 10000
E = 160000
D = 256
NH = 16
HD = 16
UW = 128


BN = 400


def _ln_relu(x, g, b):
    mu = jnp.mean(x, axis=-1, keepdims=True)
    var = jnp.mean((x - mu) ** 2, axis=-1, keepdims=True)
    x = (x - mu) * lax.rsqrt(var + 1e-5) * g + b
    return jnp.maximum(x, 0.0)


def _k1_body(h_ref, wq1_ref, bq1_ref, gq_ref, beq_ref, wq2_ref, bq2_ref,
             wk_hi_ref, wk_hj_ref, wv_hi_ref, wv_hj_ref,
             tdst_ref, tsrc_ref):
    h = h_ref[...]
    x = jnp.dot(h, wq1_ref[...], preferred_element_type=jnp.float32) + bq1_ref[...]
    x = _ln_relu(x, gq_ref[...], beq_ref[...])
    q = jnp.dot(x, wq2_ref[...], preferred_element_type=jnp.float32) + bq2_ref[...]
    tdst_ref[:, 0:D] = q
    tdst_ref[:, D:2 * D] = jnp.dot(h, wk_hi_ref[...], preferred_element_type=jnp.float32)
    tdst_ref[:, 2 * D:3 * D] = jnp.dot(h, wv_hi_ref[...], preferred_element_type=jnp.float32)
    tsrc_ref[:, 0:D] = jnp.dot(h, wk_hj_ref[...], preferred_element_type=jnp.float32)
    tsrc_ref[:, D:2 * D] = jnp.dot(h, wv_hj_ref[...], preferred_element_type=jnp.float32)


def _k1(h, wq1, bq1, gq, beq, wq2, bq2, wk_hi, wk_hj, wv_hi, wv_hj):
    nb = N // BN
    full = lambda shape: pl.BlockSpec(shape, lambda i: (0, 0))
    return pl.pallas_call(
        _k1_body,
        grid=(nb,),
        in_specs=[
            pl.BlockSpec((BN, D), lambda i: (i, 0)),
            full((D, D)), full((1, D)), full((1, D)), full((1, D)),
            full((D, D)), full((1, D)),
            full((D, D)), full((D, D)), full((D, D)), full((D, D)),
        ],
        out_specs=[
            pl.BlockSpec((BN, 3 * D), lambda i: (i, 0)),
            pl.BlockSpec((BN, 2 * D), lambda i: (i, 0)),
        ],
        out_shape=[
            jax.ShapeDtypeStruct((N, 3 * D), jnp.float32),
            jax.ShapeDtypeStruct((N, 2 * D), jnp.float32),
        ],
    )(h, wq1, bq1, gq, beq, wq2, bq2, wk_hi, wk_hj, wv_hi, wv_hj)


EPW = E // 32
CH = 40
NCH = EPW // CH


def _k2_body(tdst_hbm, tsrc_hbm, dst_hbm, src_hbm, gd_hbm, gs_hbm,
             idxd_v, idxs_v, rd0, rd1, rs0, rs1, sd0, sd1, ss0, ss1):
    wid = lax.axis_index("s") * 2 + lax.axis_index("c")
    base = wid * EPW
    pltpu.sync_copy(dst_hbm.at[pl.ds(base, EPW)], idxd_v)
    pltpu.sync_copy(src_hbm.at[pl.ds(base, EPW)], idxs_v)
    rd = (rd0, rd1)
    rs = (rs0, rs1)
    sdm = (sd0, sd1)
    ssm = (ss0, ss1)

    def start(i, b):
        pltpu.async_copy(tdst_hbm.at[idxd_v.at[pl.ds(i * CH, CH)]], rd[b], sdm[b])
        pltpu.async_copy(tsrc_hbm.at[idxs_v.at[pl.ds(i * CH, CH)]], rs[b], ssm[b])

    def wait(b):
        pltpu.make_async_copy(tdst_hbm.at[idxd_v.at[pl.ds(0, CH)]], rd[b], sdm[b]).wait()
        pltpu.make_async_copy(tsrc_hbm.at[idxs_v.at[pl.ds(0, CH)]], rs[b], ssm[b]).wait()

    def out(i, b):
        off = base + i * CH
        pltpu.sync_copy(rd[b], gd_hbm.at[pl.ds(off, CH)])
        pltpu.sync_copy(rs[b], gs_hbm.at[pl.ds(off, CH)])

    start(0, 0)
    start(1, 1)

    def step(j, c):
        for b in range(2):
            i = 2 * j + b
            wait(b)
            out(i, b)

            @pl.when(i + 2 < NCH)
            def _():
                start(i + 2, b)
        return c

    lax.fori_loop(0, (NCH - 1) // 2, step, 0)
    wait(0)
    out(NCH - 1, 0)


def _k2(tdst, tsrc, dst, src):
    mesh = plsc.VectorSubcoreMesh(core_axis_name="c", subcore_axis_name="s")
    f = pl.kernel(
        _k2_body,
        mesh=mesh,
        out_type=[
            jax.ShapeDtypeStruct((E, 384), jnp.int32),
            jax.ShapeDtypeStruct((E, 256), jnp.int32),
        ],
        scratch_types=[
            pltpu.VMEM((EPW,), jnp.int32),
            pltpu.VMEM((EPW,), jnp.int32),
            pltpu.VMEM((CH, 384), jnp.int32),
            pltpu.VMEM((CH, 384), jnp.int32),
            pltpu.VMEM((CH, 256), jnp.int32),
            pltpu.VMEM((CH, 256), jnp.int32),
            pltpu.SemaphoreType.DMA,
            pltpu.SemaphoreType.DMA,
            pltpu.SemaphoreType.DMA,
            pltpu.SemaphoreType.DMA,
        ],
    )
    return f(tdst, tsrc, dst, src)


BE = 640


def _unpack(xi32):
    lo = lax.bitcast_convert_type(xi32 << 16, jnp.float32)
    hi = lax.bitcast_convert_type(xi32 & jnp.int32(-65536), jnp.float32)
    return jnp.concatenate([lo, hi], axis=1)


def _k3_body(efrf_ref, gd_ref, gs_ref, wer_ref, wew_ref,
             bk1_ref, gk_ref, bek_ref, wk2_ref, bk2_ref,
             bv1_ref, gv_ref, bev_ref, wv2_ref, bv2_ref,
             hmat_ref, ewb_ref,
             s_ref, v2_ref, m_ref):
    efrf = efrf_ref[...]
    pk_d = _unpack(gd_ref[:, 128:256])
    pv_d = _unpack(gd_ref[:, 256:384])
    pk_s = _unpack(gs_ref[:, 0:128])
    pv_s = _unpack(gs_ref[:, 128:256])
    b16 = lambda x: x.astype(jnp.bfloat16)
    pre = jnp.dot(b16(efrf), b16(wer_ref[...]), preferred_element_type=jnp.float32)
    pre_k = pre[:, 0:D] + pk_d + pk_s + bk1_ref[...]
    pre_v = pre[:, D:2 * D] + pv_d + pv_s + bv1_ref[...]
    xk = _ln_relu(pre_k, gk_ref[...], bek_ref[...])
    xv = _ln_relu(pre_v, gv_ref[...], bev_ref[...])
    k = jnp.dot(b16(xk), b16(wk2_ref[...]), preferred_element_type=jnp.float32) + bk2_ref[...]
    v = jnp.dot(b16(xv), b16(wv2_ref[...]), preferred_element_type=jnp.float32) + bv2_ref[...]
    qd = _unpack(gd_ref[:, 0:128])
    s = jnp.dot(b16(qd * k), b16(hmat_ref[...]), preferred_element_type=jnp.float32) * 0.25
    ew_pre = jnp.dot(efrf, wew_ref[...], preferred_element_type=jnp.float32)
    e_w = jax.nn.sigmoid(ew_pre[:, 0:1] + ewb_ref[0, 0])
    s_ref[...] = s
    v2_ref[...] = v * e_w
    bmax = jnp.max(s)
    i = pl.program_id(0)

    @pl.when(i == 0)
    def _():
        m_ref[0, 0] = bmax

    @pl.when(i > 0)
    def _():
        m_ref[0, 0] = jnp.maximum(m_ref[0, 0], bmax)


def _k3(efrf, gd, gs, wer, wew, bk1, gk, bek, wk2, bk2, bv1, gv, bev, wv2, bv2, hmat, ewb):
    nb = E // BE
    full = lambda shape: pl.BlockSpec(shape, lambda i: (0, 0))
    return pl.pallas_call(
        _k3_body,
        grid=(nb,),
        in_specs=[
            pl.BlockSpec((BE, 80), lambda i: (i, 0)),
            pl.BlockSpec((BE, 384), lambda i: (i, 0)),
            pl.BlockSpec((BE, 256), lambda i: (i, 0)),
            full((80, 2 * D)), full((80, 8)),
            full((1, D)), full((1, D)), full((1, D)), full((D, D)), full((1, D)),
            full((1, D)), full((1, D)), full((1, D)), full((D, NH)), full((1, NH)),
            full((D, NH)), full((1, 1)),
        ],
        out_specs=[
            pl.BlockSpec((BE, NH), lambda i: (i, 0)),
            pl.BlockSpec((BE, NH), lambda i: (i, 0)),
            pl.BlockSpec((1, 1), lambda i: (0, 0), memory_space=pltpu.SMEM),
        ],
        out_shape=[
            jax.ShapeDtypeStruct((E, NH), jnp.float32),
            jax.ShapeDtypeStruct((E, NH), jnp.float32),
            jax.ShapeDtypeStruct((1, 1), jnp.float32),
        ],
    )(efrf, gd, gs, wer, wew, bk1, gk, bek, wk2, bk2, bv1, gv, bev, wv2, bv2, hmat, ewb)


def _k4_body(s_ref, v2_ref, rx_ref, m_ref, u_ref):
    t = jnp.exp(s_ref[...] - m_ref[0, 0])
    p = t * v2_ref[...]
    rx = rx_ref[...]
    u_ref[:, 0:NH] = t
    u_ref[:, NH:2 * NH] = p * rx[:, 0:1]
    u_ref[:, 2 * NH:3 * NH] = p * rx[:, 1:2]
    u_ref[:, 3 * NH:4 * NH] = p * rx[:, 2:3]
    u_ref[:, 4 * NH:UW] = jnp.zeros((BE, UW - 4 * NH), jnp.float32)


def _k4(s, v2, rel_x, m):
    nb = E // BE
    return pl.pallas_call(
        _k4_body,
        grid=(nb,),
        in_specs=[
            pl.BlockSpec((BE, NH), lambda i: (i, 0)),
            pl.BlockSpec((BE, NH), lambda i: (i, 0)),
            pl.BlockSpec((BE, 3), lambda i: (i, 0)),
            pl.BlockSpec((1, 1), lambda i: (0, 0), memory_space=pltpu.SMEM),
        ],
        out_specs=pl.BlockSpec((BE, UW), lambda i: (i, 0)),
        out_shape=jax.ShapeDtypeStruct((E, UW), jnp.float32),
    )(s, v2, rel_x, m)


def _k5_body(u_hbm, dst_hbm, zeros_hbm, s_hbm, idx_v, u_v, shared):
    cid = lax.axis_index("c")
    sid = lax.axis_index("s")
    base = (cid * 16 + sid) * EPW

    @pl.when(sid == 0)
    def _():
        pltpu.sync_copy(zeros_hbm, shared)

    plsc.subcore_barrier()

    def step(i, carry):
        off = base + i * CH
        pltpu.sync_copy(dst_hbm.at[pl.ds(off, CH)], idx_v)
        pltpu.sync_copy(u_hbm.at[pl.ds(off, CH)], u_v)
        pltpu.sync_copy(u_v, shared.at[idx_v], add=True)
        return carry

    lax.fori_loop(0, NCH, step, 0)
    plsc.subcore_barrier()

    @pl.when(sid == 0)
    def _():
        pltpu.sync_copy(shared, s_hbm.at[cid])


def _k5(u, dst, zeros):
    mesh = plsc.VectorSubcoreMesh(core_axis_name="c", subcore_axis_name="s")
    f = pl.kernel(
        _k5_body,
        mesh=mesh,
        out_type=jax.ShapeDtypeStruct((2, N, UW), jnp.float32),
        scratch_types=[
            pltpu.VMEM((CH,), jnp.int32),
            pltpu.VMEM((CH, UW), jnp.float32),
            pltpu.VMEM_SHARED((N, UW), jnp.float32),
        ],
    )
    return f(u, dst, zeros)


BN6 = 2000


def _k6_body(s_ref, o_ref):
    den = s_ref[0, :, 0:NH] + s_ref[1, :, 0:NH]
    r = 1.0 / (den + 1e-16)
    for c in range(3):
        a = s_ref[0, :, (c + 1) * NH:(c + 2) * NH] + s_ref[1, :, (c + 1) * NH:(c + 2) * NH]
        o_ref[:, c:c + 1] = jnp.sum(a * r, axis=-1, keepdims=True) * (1.0 / NH)
    o_ref[:, 3:4] = jnp.zeros((BN6, 1), jnp.float32)


def _k6(s):
    nb = N // BN6
    return pl.pallas_call(
        _k6_body,
        grid=(nb,),
        in_specs=[pl.BlockSpec((2, BN6, UW), lambda i: (0, i, 0))],
        out_specs=pl.BlockSpec((BN6, 4), lambda i: (i, 0)),
        out_shape=jax.ShapeDtypeStruct((N, 4), jnp.float32),
    )(s)


def kernel(h, rel_x, r_feat, edge_feat, edge_index, xk_W1, xk_b1, xk_g, xk_be,
           xk_W2, xk_b2, xv_W1, xv_b1, xv_g, xv_be, xv_W2, xv_b2,
           xq_W1, xq_b1, xq_g, xq_be, xq_W2, xq_b2, ew_W, ew_b):
    src = edge_index[0]
    dst = edge_index[1]
    row = lambda x: x.reshape(1, -1)

    tdst, tsrc = _k1(
        h, xq_W1, row(xq_b1), row(xq_g), row(xq_be), xq_W2, row(xq_b2),
        xk_W1[80:336], xk_W1[336:592], xv_W1[80:336], xv_W1[336:592])

    def pack_block(b):
        b16 = lax.bitcast_convert_type(b.astype(jnp.bfloat16), jnp.uint16)
        lo = b16[:, :128].astype(jnp.uint32)
        hi = b16[:, 128:].astype(jnp.uint32)
        return lax.bitcast_convert_type(lo | (hi << 16), jnp.int32)

    tdst_p = jnp.concatenate([pack_block(tdst[:, i * D:(i + 1) * D]) for i in range(3)], 1)
    tsrc_p = jnp.concatenate([pack_block(tsrc[:, i * D:(i + 1) * D]) for i in range(2)], 1)
    gd, gs = _k2(tdst_p, tsrc_p, dst, src)

    efrf = jnp.concatenate([edge_feat, r_feat], axis=1)
    wer = jnp.concatenate([xk_W1[0:80], xv_W1[0:80]], axis=1)
    wew = jnp.zeros((80, 8), jnp.float32).at[16:80, 0].set(ew_W[:, 0])
    hmat = (jax.lax.broadcasted_iota(jnp.int32, (D, NH), 0) // HD
            == jax.lax.broadcasted_iota(jnp.int32, (D, NH), 1)).astype(jnp.float32)

    s, v2, m = _k3(efrf, gd, gs, wer, wew,
                   row(xk_b1), row(xk_g), row(xk_be), xk_W2, row(xk_b2),
                   row(xv_b1), row(xv_g), row(xv_be), xv_W2, row(xv_b2),
                   hmat, ew_b.reshape(1, 1))

    u = _k4(s, v2, rel_x, m)

    sacc = _k5(u, dst, jnp.zeros((N, UW), jnp.float32))

    out = _k6(sacc)
    return out[:, :3]


# speedup vs baseline: 17.3822x; 1.0875x over previous
import jax
import jax.numpy as jnp
from jax import lax
from jax.experimental import pallas as pl
from jax.experimental.pallas import tpu as pltpu
from jax.experimental.pallas import tpu_sc as plsc

N = 10000
E = 160000
D = 256
NH = 16
HD = 16
UW = 128


BN = 400


def _ln_relu(x, g, b):
    mu = jnp.mean(x, axis=-1, keepdims=True)
    var = jnp.mean((x - mu) ** 2, axis=-1, keepdims=True)
    x = (x - mu) * lax.rsqrt(var + 1e-5) * g + b
    return jnp.maximum(x, 0.0)


def _k1_body(h_ref, wq1_ref, bq1_ref, gq_ref, beq_ref, wq2_ref, bq2_ref,
             wk_hi_ref, wk_hj_ref, wv_hi_ref, wv_hj_ref,
             tdst_ref, tsrc_ref):
    h = h_ref[...]
    x = jnp.dot(h, wq1_ref[...], preferred_element_type=jnp.float32) + bq1_ref[...]
    x = _ln_relu(x, gq_ref[...], beq_ref[...])
    q = jnp.dot(x, wq2_ref[...], preferred_element_type=jnp.float32) + bq2_ref[...]
    tdst_ref[:, 0:D] = q
    tdst_ref[:, D:2 * D] = jnp.dot(h, wk_hi_ref[...], preferred_element_type=jnp.float32)
    tdst_ref[:, 2 * D:3 * D] = jnp.dot(h, wv_hi_ref[...], preferred_element_type=jnp.float32)
    tsrc_ref[:, 0:D] = jnp.dot(h, wk_hj_ref[...], preferred_element_type=jnp.float32)
    tsrc_ref[:, D:2 * D] = jnp.dot(h, wv_hj_ref[...], preferred_element_type=jnp.float32)


def _k1(h, wq1, bq1, gq, beq, wq2, bq2, wk_hi, wk_hj, wv_hi, wv_hj):
    nb = N // BN
    full = lambda shape: pl.BlockSpec(shape, lambda i: (0, 0))
    return pl.pallas_call(
        _k1_body,
        grid=(nb,),
        in_specs=[
            pl.BlockSpec((BN, D), lambda i: (i, 0)),
            full((D, D)), full((1, D)), full((1, D)), full((1, D)),
            full((D, D)), full((1, D)),
            full((D, D)), full((D, D)), full((D, D)), full((D, D)),
        ],
        out_specs=[
            pl.BlockSpec((BN, 3 * D), lambda i: (i, 0)),
            pl.BlockSpec((BN, 2 * D), lambda i: (i, 0)),
        ],
        out_shape=[
            jax.ShapeDtypeStruct((N, 3 * D), jnp.float32),
            jax.ShapeDtypeStruct((N, 2 * D), jnp.float32),
        ],
    )(h, wq1, bq1, gq, beq, wq2, bq2, wk_hi, wk_hj, wv_hi, wv_hj)


EPW = E // 32
CH = 40
NCH = EPW // CH
G_CH = 96
G_N = EPW // G_CH
G_TAIL = EPW - G_N * G_CH


def _k2_body(tdst_hbm, tsrc_hbm, dst_hbm, src_hbm, gd_hbm, gs_hbm,
             id0, id1, is0, is1, rd0, rd1, rs0, rs1, sd0, sd1, ss0, ss1):
    wid = lax.axis_index("s") * 2 + lax.axis_index("c")
    base = wid * EPW
    idb = (id0, id1)
    isb = (is0, is1)
    rd = (rd0, rd1)
    rs = (rs0, rs1)
    sdm = (sd0, sd1)
    ssm = (ss0, ss1)

    def start(off, n, b):
        pltpu.sync_copy(dst_hbm.at[pl.ds(off, G_CH)], idb[b])
        pltpu.sync_copy(src_hbm.at[pl.ds(off, G_CH)], isb[b])
        pltpu.async_copy(tdst_hbm.at[idb[b]], rd[b], sdm[b])
        pltpu.async_copy(tsrc_hbm.at[isb[b]], rs[b], ssm[b])

    def wait(b):
        pltpu.make_async_copy(tdst_hbm.at[idb[b]], rd[b], sdm[b]).wait()
        pltpu.make_async_copy(tsrc_hbm.at[isb[b]], rs[b], ssm[b]).wait()

    def out(off, b):
        pltpu.sync_copy(rd[b], gd_hbm.at[pl.ds(off, G_CH)])
        pltpu.sync_copy(rs[b], gs_hbm.at[pl.ds(off, G_CH)])

    start(base, G_CH, 0)
    start(base + G_CH, G_CH, 1)

    def step(j, c):
        for b in range(2):
            i = 2 * j + b
            wait(b)
            out(base + i * G_CH, b)

            @pl.when(i + 2 < G_N)
            def _():
                start(base + (i + 2) * G_CH, G_CH, b)
        return c

    lax.fori_loop(0, G_N // 2, step, 0)
    toff = base + G_N * G_CH
    pltpu.sync_copy(dst_hbm.at[pl.ds(toff, G_TAIL)], id0.at[pl.ds(0, G_TAIL)])
    pltpu.sync_copy(src_hbm.at[pl.ds(toff, G_TAIL)], is0.at[pl.ds(0, G_TAIL)])
    pltpu.async_copy(tdst_hbm.at[id0.at[pl.ds(0, G_TAIL)]], rd0.at[pl.ds(0, G_TAIL)], sd0).wait()
    pltpu.async_copy(tsrc_hbm.at[is0.at[pl.ds(0, G_TAIL)]], rs0.at[pl.ds(0, G_TAIL)], ss0).wait()
    pltpu.sync_copy(rd0.at[pl.ds(0, G_TAIL)], gd_hbm.at[pl.ds(toff, G_TAIL)])
    pltpu.sync_copy(rs0.at[pl.ds(0, G_TAIL)], gs_hbm.at[pl.ds(toff, G_TAIL)])


def _k2(tdst, tsrc, dst, src):
    mesh = plsc.VectorSubcoreMesh(core_axis_name="c", subcore_axis_name="s")
    f = pl.kernel(
        _k2_body,
        mesh=mesh,
        out_type=[
            jax.ShapeDtypeStruct((E, 384), jnp.int32),
            jax.ShapeDtypeStruct((E, 256), jnp.int32),
        ],
        scratch_types=[
            pltpu.VMEM((G_CH,), jnp.int32),
            pltpu.VMEM((G_CH,), jnp.int32),
            pltpu.VMEM((G_CH,), jnp.int32),
            pltpu.VMEM((G_CH,), jnp.int32),
            pltpu.VMEM((G_CH, 384), jnp.int32),
            pltpu.VMEM((G_CH, 384), jnp.int32),
            pltpu.VMEM((G_CH, 256), jnp.int32),
            pltpu.VMEM((G_CH, 256), jnp.int32),
            pltpu.SemaphoreType.DMA,
            pltpu.SemaphoreType.DMA,
            pltpu.SemaphoreType.DMA,
            pltpu.SemaphoreType.DMA,
        ],
    )
    return f(tdst, tsrc, dst, src)


BE = 640


def _unpack(xi32):
    lo = lax.bitcast_convert_type(xi32 << 16, jnp.float32)
    hi = lax.bitcast_convert_type(xi32 & jnp.int32(-65536), jnp.float32)
    return jnp.concatenate([lo, hi], axis=1)


def _k3_body(ef_ref, rf_ref, gd_ref, gs_ref, wer_e_ref, wer_r_ref, wew_ref,
             bk1_ref, gk_ref, bek_ref, wk2_ref, bk2_ref,
             bv1_ref, gv_ref, bev_ref, wv2_ref, bv2_ref,
             hmat_ref, ewb_ref,
             s_ref, v2_ref, m_ref):
    ef = ef_ref[...]
    rf = rf_ref[...]
    pk_d = _unpack(gd_ref[:, 128:256])
    pv_d = _unpack(gd_ref[:, 256:384])
    pk_s = _unpack(gs_ref[:, 0:128])
    pv_s = _unpack(gs_ref[:, 128:256])
    b16 = lambda x: x.astype(jnp.bfloat16)
    pre = (jnp.dot(b16(ef), b16(wer_e_ref[...]), preferred_element_type=jnp.float32)
           + jnp.dot(b16(rf), b16(wer_r_ref[...]), preferred_element_type=jnp.float32))
    pre_k = pre[:, 0:D] + pk_d + pk_s + bk1_ref[...]
    pre_v = pre[:, D:2 * D] + pv_d + pv_s + bv1_ref[...]
    xk = _ln_relu(pre_k, gk_ref[...], bek_ref[...])
    xv = _ln_relu(pre_v, gv_ref[...], bev_ref[...])
    k = jnp.dot(b16(xk), b16(wk2_ref[...]), preferred_element_type=jnp.float32) + bk2_ref[...]
    v = jnp.dot(b16(xv), b16(wv2_ref[...]), preferred_element_type=jnp.float32) + bv2_ref[...]
    qd = _unpack(gd_ref[:, 0:128])
    s = jnp.dot(b16(qd * k), b16(hmat_ref[...]), preferred_element_type=jnp.float32) * 0.25
    ew_pre = jnp.dot(rf, wew_ref[...], preferred_element_type=jnp.float32)
    e_w = jax.nn.sigmoid(ew_pre[:, 0:1] + ewb_ref[0, 0])
    s_ref[...] = s
    v2_ref[...] = v * e_w
    bmax = jnp.max(s)
    i = pl.program_id(0)

    @pl.when(i == 0)
    def _():
        m_ref[0, 0] = bmax

    @pl.when(i > 0)
    def _():
        m_ref[0, 0] = jnp.maximum(m_ref[0, 0], bmax)


def _k3(ef, rf, gd, gs, wer_e, wer_r, wew, bk1, gk, bek, wk2, bk2, bv1, gv, bev, wv2, bv2, hmat, ewb):
    nb = E // BE
    full = lambda shape: pl.BlockSpec(shape, lambda i: (0, 0))
    return pl.pallas_call(
        _k3_body,
        grid=(nb,),
        in_specs=[
            pl.BlockSpec((BE, 16), lambda i: (i, 0)),
            pl.BlockSpec((BE, 64), lambda i: (i, 0)),
            pl.BlockSpec((BE, 384), lambda i: (i, 0)),
            pl.BlockSpec((BE, 256), lambda i: (i, 0)),
            full((16, 2 * D)), full((64, 2 * D)), full((64, 8)),
            full((1, D)), full((1, D)), full((1, D)), full((D, D)), full((1, D)),
            full((1, D)), full((1, D)), full((1, D)), full((D, NH)), full((1, NH)),
            full((D, NH)), full((1, 1)),
        ],
        out_specs=[
            pl.BlockSpec((BE, NH), lambda i: (i, 0)),
            pl.BlockSpec((BE, NH), lambda i: (i, 0)),
            pl.BlockSpec((1, 1), lambda i: (0, 0), memory_space=pltpu.SMEM),
        ],
        out_shape=[
            jax.ShapeDtypeStruct((E, NH), jnp.float32),
            jax.ShapeDtypeStruct((E, NH), jnp.float32),
            jax.ShapeDtypeStruct((1, 1), jnp.float32),
        ],
    )(ef, rf, gd, gs, wer_e, wer_r, wew, bk1, gk, bek, wk2, bk2, bv1, gv, bev, wv2, bv2, hmat, ewb)


def _k4_body(s_ref, v2_ref, rx_ref, m_ref, u_ref):
    t = jnp.exp(s_ref[...] - m_ref[0, 0])
    p = t * v2_ref[...]
    rx = rx_ref[...]
    u_ref[:, 0:NH] = t
    u_ref[:, NH:2 * NH] = p * rx[:, 0:1]
    u_ref[:, 2 * NH:3 * NH] = p * rx[:, 1:2]
    u_ref[:, 3 * NH:4 * NH] = p * rx[:, 2:3]
    u_ref[:, 4 * NH:UW] = jnp.zeros((BE, UW - 4 * NH), jnp.float32)


def _k4(s, v2, rel_x, m):
    nb = E // BE
    return pl.pallas_call(
        _k4_body,
        grid=(nb,),
        in_specs=[
            pl.BlockSpec((BE, NH), lambda i: (i, 0)),
            pl.BlockSpec((BE, NH), lambda i: (i, 0)),
            pl.BlockSpec((BE, 3), lambda i: (i, 0)),
            pl.BlockSpec((1, 1), lambda i: (0, 0), memory_space=pltpu.SMEM),
        ],
        out_specs=pl.BlockSpec((BE, UW), lambda i: (i, 0)),
        out_shape=jax.ShapeDtypeStruct((E, UW), jnp.float32),
    )(s, v2, rel_x, m)


SCH = 40
SNCH = EPW // SCH


def _k5_body(u_hbm, dst2_hbm, zeros_hbm, s_hbm, slab_v, u0, u1, su0, su1, shared):
    cid = lax.axis_index("c")
    sid = lax.axis_index("s")
    wid = cid * 16 + sid
    base = wid * EPW

    @pl.when(sid == 0)
    def _():
        pltpu.sync_copy(zeros_hbm, shared)

    pltpu.sync_copy(dst2_hbm.at[wid], slab_v)
    plsc.subcore_barrier()

    ub = (u0, u1)
    sub = (su0, su1)

    def start(i, b):
        pltpu.async_copy(u_hbm.at[pl.ds(base + i * SCH, SCH)], ub[b], sub[b])

    def wait(b):
        pltpu.make_async_copy(u_hbm.at[pl.ds(base, SCH)], ub[b], sub[b]).wait()

    start(0, 0)
    start(1, 1)

    def step(j, c):
        for b in range(2):
            i = 2 * j + b
            wait(b)
            pltpu.sync_copy(ub[b], shared.at[slab_v.at[i]], add=True)

            @pl.when(i + 2 < SNCH)
            def _():
                start(i + 2, b)
        return c

    lax.fori_loop(0, SNCH // 2, step, 0)
    wait(0)
    pltpu.sync_copy(ub[0], shared.at[slab_v.at[SNCH - 1]], add=True)
    plsc.subcore_barrier()

    @pl.when(sid == 0)
    def _():
        pltpu.sync_copy(shared, s_hbm.at[cid])


def _k5(u, dst2, zeros):
    mesh = plsc.VectorSubcoreMesh(core_axis_name="c", subcore_axis_name="s")
    f = pl.kernel(
        _k5_body,
        mesh=mesh,
        out_type=jax.ShapeDtypeStruct((2, N, UW), jnp.float32),
        scratch_types=[
            pltpu.VMEM((SNCH, SCH), jnp.int32),
            pltpu.VMEM((SCH, UW), jnp.float32),
            pltpu.VMEM((SCH, UW), jnp.float32),
            pltpu.SemaphoreType.DMA,
            pltpu.SemaphoreType.DMA,
            pltpu.VMEM_SHARED((N, UW), jnp.float32),
        ],
    )
    return f(u, dst2, zeros)


BN6 = 2000


def _k6_body(s_ref, o_ref):
    den = s_ref[0, :, 0:NH] + s_ref[1, :, 0:NH]
    r = 1.0 / (den + 1e-16)
    for c in range(3):
        a = s_ref[0, :, (c + 1) * NH:(c + 2) * NH] + s_ref[1, :, (c + 1) * NH:(c + 2) * NH]
        o_ref[:, c:c + 1] = jnp.sum(a * r, axis=-1, keepdims=True) * (1.0 / NH)
    o_ref[:, 3:4] = jnp.zeros((BN6, 1), jnp.float32)


def _k6(s):
    nb = N // BN6
    return pl.pallas_call(
        _k6_body,
        grid=(nb,),
        in_specs=[pl.BlockSpec((2, BN6, UW), lambda i: (0, i, 0))],
        out_specs=pl.BlockSpec((BN6, 4), lambda i: (i, 0)),
        out_shape=jax.ShapeDtypeStruct((N, 4), jnp.float32),
    )(s)


def kernel(h, rel_x, r_feat, edge_feat, edge_index, xk_W1, xk_b1, xk_g, xk_be,
           xk_W2, xk_b2, xv_W1, xv_b1, xv_g, xv_be, xv_W2, xv_b2,
           xq_W1, xq_b1, xq_g, xq_be, xq_W2, xq_b2, ew_W, ew_b):
    src = edge_index[0]
    dst = edge_index[1]
    row = lambda x: x.reshape(1, -1)

    tdst, tsrc = _k1(
        h, xq_W1, row(xq_b1), row(xq_g), row(xq_be), xq_W2, row(xq_b2),
        xk_W1[80:336], xk_W1[336:592], xv_W1[80:336], xv_W1[336:592])

    def pack_block(b):
        b16 = lax.bitcast_convert_type(b.astype(jnp.bfloat16), jnp.uint16)
        lo = b16[:, :128].astype(jnp.uint32)
        hi = b16[:, 128:].astype(jnp.uint32)
        return lax.bitcast_convert_type(lo | (hi << 16), jnp.int32)

    tdst_p = jnp.concatenate([pack_block(tdst[:, i * D:(i + 1) * D]) for i in range(3)], 1)
    tsrc_p = jnp.concatenate([pack_block(tsrc[:, i * D:(i + 1) * D]) for i in range(2)], 1)
    gd, gs = _k2(tdst_p, tsrc_p, dst, src)

    wer_e = jnp.concatenate([xk_W1[0:16], xv_W1[0:16]], axis=1)
    wer_r = jnp.concatenate([xk_W1[16:80], xv_W1[16:80]], axis=1)
    wew = jnp.zeros((64, 8), jnp.float32).at[:, 0].set(ew_W[:, 0])
    hmat = (jax.lax.broadcasted_iota(jnp.int32, (D, NH), 0) // HD
            == jax.lax.broadcasted_iota(jnp.int32, (D, NH), 1)).astype(jnp.float32)

    s, v2, m = _k3(edge_feat, r_feat, gd, gs, wer_e, wer_r, wew,
                   row(xk_b1), row(xk_g), row(xk_be), xk_W2, row(xk_b2),
                   row(xv_b1), row(xv_g), row(xv_be), xv_W2, row(xv_b2),
                   hmat, ew_b.reshape(1, 1))

    u = _k4(s, v2, rel_x, m)

    sacc = _k5(u, dst.reshape(32, SNCH, SCH), jnp.zeros((N, UW), jnp.float32))

    out = _k6(sacc)
    return out[:, :3]


# speedup vs baseline: 17.5778x; 1.0113x over previous
import jax
import jax.numpy as jnp
from jax import lax
from jax.experimental import pallas as pl
from jax.experimental.pallas import tpu as pltpu
from jax.experimental.pallas import tpu_sc as plsc

N = 10000
E = 160000
D = 256
NH = 16
HD = 16
UW = 128


BN = 400


def _ln_relu(x, g, b):
    mu = jnp.mean(x, axis=-1, keepdims=True)
    var = jnp.mean((x - mu) ** 2, axis=-1, keepdims=True)
    x = (x - mu) * lax.rsqrt(var + 1e-5) * g + b
    return jnp.maximum(x, 0.0)


def _k1_body(h_ref, wq1_ref, bq1_ref, gq_ref, beq_ref, wq2_ref, bq2_ref,
             wk_hi_ref, wk_hj_ref, wv_hi_ref, wv_hj_ref,
             tdst_ref, tsrc_ref):
    h = h_ref[...]
    x = jnp.dot(h, wq1_ref[...], preferred_element_type=jnp.float32) + bq1_ref[...]
    x = _ln_relu(x, gq_ref[...], beq_ref[...])
    q = jnp.dot(x, wq2_ref[...], preferred_element_type=jnp.float32) + bq2_ref[...]
    tdst_ref[:, 0:D] = q
    tdst_ref[:, D:2 * D] = jnp.dot(h, wk_hi_ref[...], preferred_element_type=jnp.float32)
    tdst_ref[:, 2 * D:3 * D] = jnp.dot(h, wv_hi_ref[...], preferred_element_type=jnp.float32)
    tsrc_ref[:, 0:D] = jnp.dot(h, wk_hj_ref[...], preferred_element_type=jnp.float32)
    tsrc_ref[:, D:2 * D] = jnp.dot(h, wv_hj_ref[...], preferred_element_type=jnp.float32)


def _k1(h, wq1, bq1, gq, beq, wq2, bq2, wk_hi, wk_hj, wv_hi, wv_hj):
    nb = N // BN
    full = lambda shape: pl.BlockSpec(shape, lambda i: (0, 0))
    return pl.pallas_call(
        _k1_body,
        grid=(nb,),
        in_specs=[
            pl.BlockSpec((BN, D), lambda i: (i, 0)),
            full((D, D)), full((1, D)), full((1, D)), full((1, D)),
            full((D, D)), full((1, D)),
            full((D, D)), full((D, D)), full((D, D)), full((D, D)),
        ],
        out_specs=[
            pl.BlockSpec((BN, 3 * D), lambda i: (i, 0)),
            pl.BlockSpec((BN, 2 * D), lambda i: (i, 0)),
        ],
        out_shape=[
            jax.ShapeDtypeStruct((N, 3 * D), jnp.float32),
            jax.ShapeDtypeStruct((N, 2 * D), jnp.float32),
        ],
    )(h, wq1, bq1, gq, beq, wq2, bq2, wk_hi, wk_hj, wv_hi, wv_hj)


EPW = E // 32
CH = 40
NCH = EPW // CH
G_CH = 96
G_N = EPW // G_CH
G_TAIL = EPW - G_N * G_CH


def _k2_body(tdst_hbm, tsrc_hbm, dst_hbm, src_hbm, gd_hbm, gs_hbm,
             id0, id1, is0, is1, rd0, rd1, rs0, rs1, sd0, sd1, ss0, ss1):
    wid = lax.axis_index("s") * 2 + lax.axis_index("c")
    base = wid * EPW
    idb = (id0, id1)
    isb = (is0, is1)
    rd = (rd0, rd1)
    rs = (rs0, rs1)
    sdm = (sd0, sd1)
    ssm = (ss0, ss1)

    def start(off, n, b):
        pltpu.sync_copy(dst_hbm.at[pl.ds(off, G_CH)], idb[b])
        pltpu.sync_copy(src_hbm.at[pl.ds(off, G_CH)], isb[b])
        pltpu.async_copy(tdst_hbm.at[idb[b]], rd[b], sdm[b])
        pltpu.async_copy(tsrc_hbm.at[isb[b]], rs[b], ssm[b])

    def wait(b):
        pltpu.make_async_copy(tdst_hbm.at[idb[b]], rd[b], sdm[b]).wait()
        pltpu.make_async_copy(tsrc_hbm.at[isb[b]], rs[b], ssm[b]).wait()

    def out(off, b):
        pltpu.sync_copy(rd[b], gd_hbm.at[pl.ds(off, G_CH)])
        pltpu.sync_copy(rs[b], gs_hbm.at[pl.ds(off, G_CH)])

    start(base, G_CH, 0)
    start(base + G_CH, G_CH, 1)

    def step(j, c):
        for b in range(2):
            i = 2 * j + b
            wait(b)
            out(base + i * G_CH, b)

            @pl.when(i + 2 < G_N)
            def _():
                start(base + (i + 2) * G_CH, G_CH, b)
        return c

    lax.fori_loop(0, G_N // 2, step, 0)
    toff = base + G_N * G_CH
    pltpu.sync_copy(dst_hbm.at[pl.ds(toff, G_TAIL)], id0.at[pl.ds(0, G_TAIL)])
    pltpu.sync_copy(src_hbm.at[pl.ds(toff, G_TAIL)], is0.at[pl.ds(0, G_TAIL)])
    pltpu.async_copy(tdst_hbm.at[id0.at[pl.ds(0, G_TAIL)]], rd0.at[pl.ds(0, G_TAIL)], sd0).wait()
    pltpu.async_copy(tsrc_hbm.at[is0.at[pl.ds(0, G_TAIL)]], rs0.at[pl.ds(0, G_TAIL)], ss0).wait()
    pltpu.sync_copy(rd0.at[pl.ds(0, G_TAIL)], gd_hbm.at[pl.ds(toff, G_TAIL)])
    pltpu.sync_copy(rs0.at[pl.ds(0, G_TAIL)], gs_hbm.at[pl.ds(toff, G_TAIL)])


def _k2(tdst, tsrc, dst, src):
    mesh = plsc.VectorSubcoreMesh(core_axis_name="c", subcore_axis_name="s")
    f = pl.kernel(
        _k2_body,
        mesh=mesh,
        out_type=[
            jax.ShapeDtypeStruct((E, 384), jnp.int32),
            jax.ShapeDtypeStruct((E, 256), jnp.int32),
        ],
        scratch_types=[
            pltpu.VMEM((G_CH,), jnp.int32),
            pltpu.VMEM((G_CH,), jnp.int32),
            pltpu.VMEM((G_CH,), jnp.int32),
            pltpu.VMEM((G_CH,), jnp.int32),
            pltpu.VMEM((G_CH, 384), jnp.int32),
            pltpu.VMEM((G_CH, 384), jnp.int32),
            pltpu.VMEM((G_CH, 256), jnp.int32),
            pltpu.VMEM((G_CH, 256), jnp.int32),
            pltpu.SemaphoreType.DMA,
            pltpu.SemaphoreType.DMA,
            pltpu.SemaphoreType.DMA,
            pltpu.SemaphoreType.DMA,
        ],
    )
    return f(tdst, tsrc, dst, src)


BE = 640


def _unpack(xi32):
    lo = lax.bitcast_convert_type(xi32 << 16, jnp.float32)
    hi = lax.bitcast_convert_type(xi32 & jnp.int32(-65536), jnp.float32)
    return jnp.concatenate([lo, hi], axis=1)


def _k3_body(ef_ref, rf_ref, gd_ref, gs_ref, wer_e_ref, wer_r_ref, wew_ref,
             bk1_ref, gk_ref, bek_ref, wk2_ref, bk2_ref,
             bv1_ref, gv_ref, bev_ref, wv2_ref, bv2_ref,
             hmat_ref, ewb_ref,
             sv_ref, m_ref):
    ef = ef_ref[...]
    rf = rf_ref[...]
    pk_d = _unpack(gd_ref[:, 128:256])
    pv_d = _unpack(gd_ref[:, 256:384])
    pk_s = _unpack(gs_ref[:, 0:128])
    pv_s = _unpack(gs_ref[:, 128:256])
    b16 = lambda x: x.astype(jnp.bfloat16)
    pre = (jnp.dot(b16(ef), b16(wer_e_ref[...]), preferred_element_type=jnp.float32)
           + jnp.dot(b16(rf), b16(wer_r_ref[...]), preferred_element_type=jnp.float32))
    pre_k = pre[:, 0:D] + pk_d + pk_s + bk1_ref[...]
    pre_v = pre[:, D:2 * D] + pv_d + pv_s + bv1_ref[...]
    xk = _ln_relu(pre_k, gk_ref[...], bek_ref[...])
    xv = _ln_relu(pre_v, gv_ref[...], bev_ref[...])
    k = jnp.dot(b16(xk), b16(wk2_ref[...]), preferred_element_type=jnp.float32) + bk2_ref[...]
    v = jnp.dot(b16(xv), b16(wv2_ref[...]), preferred_element_type=jnp.float32) + bv2_ref[...]
    qd = _unpack(gd_ref[:, 0:128])
    s = jnp.dot(b16(qd * k), b16(hmat_ref[...]), preferred_element_type=jnp.float32) * 0.25
    ew_pre = jnp.dot(rf, wew_ref[...], preferred_element_type=jnp.float32)
    e_w = jax.nn.sigmoid(ew_pre[:, 0:1] + ewb_ref[0, 0])
    sv_ref[:, 0:NH] = s
    sv_ref[:, NH:2 * NH] = v * e_w
    bmax = jnp.max(s)
    i = pl.program_id(0)

    @pl.when(i == 0)
    def _():
        m_ref[0, 0] = bmax

    @pl.when(i > 0)
    def _():
        m_ref[0, 0] = jnp.maximum(m_ref[0, 0], bmax)


def _k3(ef, rf, gd, gs, wer_e, wer_r, wew, bk1, gk, bek, wk2, bk2, bv1, gv, bev, wv2, bv2, hmat, ewb):
    nb = E // BE
    full = lambda shape: pl.BlockSpec(shape, lambda i: (0, 0))
    return pl.pallas_call(
        _k3_body,
        grid=(nb,),
        in_specs=[
            pl.BlockSpec((BE, 16), lambda i: (i, 0)),
            pl.BlockSpec((BE, 64), lambda i: (i, 0)),
            pl.BlockSpec((BE, 384), lambda i: (i, 0)),
            pl.BlockSpec((BE, 256), lambda i: (i, 0)),
            full((16, 2 * D)), full((64, 2 * D)), full((64, 8)),
            full((1, D)), full((1, D)), full((1, D)), full((D, D)), full((1, D)),
            full((1, D)), full((1, D)), full((1, D)), full((D, NH)), full((1, NH)),
            full((D, NH)), full((1, 1)),
        ],
        out_specs=[
            pl.BlockSpec((BE, 2 * NH), lambda i: (i, 0)),
            pl.BlockSpec((1, 1), lambda i: (0, 0), memory_space=pltpu.SMEM),
        ],
        out_shape=[
            jax.ShapeDtypeStruct((E, 2 * NH), jnp.float32),
            jax.ShapeDtypeStruct((1, 1), jnp.float32),
        ],
    )(ef, rf, gd, gs, wer_e, wer_r, wew, bk1, gk, bek, wk2, bk2, bv1, gv, bev, wv2, bv2, hmat, ewb)


def _k4_body(sv_ref, rx_ref, m_ref, u_ref):
    t = jnp.exp(sv_ref[:, 0:NH] - m_ref[0, 0])
    p = t * sv_ref[:, NH:2 * NH]
    rx = rx_ref[...]
    u_ref[:, 0:NH] = t
    u_ref[:, NH:2 * NH] = p * rx[:, 0:1]
    u_ref[:, 2 * NH:3 * NH] = p * rx[:, 1:2]
    u_ref[:, 3 * NH:4 * NH] = p * rx[:, 2:3]
    u_ref[:, 4 * NH:UW] = jnp.zeros((BE, UW - 4 * NH), jnp.float32)


def _k4(sv, rel_x, m):
    nb = E // BE
    return pl.pallas_call(
        _k4_body,
        grid=(nb,),
        in_specs=[
            pl.BlockSpec((BE, 2 * NH), lambda i: (i, 0)),
            pl.BlockSpec((BE, 3), lambda i: (i, 0)),
            pl.BlockSpec((1, 1), lambda i: (0, 0), memory_space=pltpu.SMEM),
        ],
        out_specs=pl.BlockSpec((BE, UW), lambda i: (i, 0)),
        out_shape=jax.ShapeDtypeStruct((E, UW), jnp.float32),
    )(sv, rel_x, m)


SCH = 40
SNCH = EPW // SCH


def _k5_body(u_hbm, dst2_hbm, zeros_hbm, s_hbm, slab_v, u0, u1, su0, su1, shared):
    cid = lax.axis_index("c")
    sid = lax.axis_index("s")
    wid = cid * 16 + sid
    base = wid * EPW

    @pl.when(sid == 0)
    def _():
        pltpu.sync_copy(zeros_hbm, shared)

    pltpu.sync_copy(dst2_hbm.at[wid], slab_v)
    plsc.subcore_barrier()

    ub = (u0, u1)
    sub = (su0, su1)

    def start(i, b):
        pltpu.async_copy(u_hbm.at[pl.ds(base + i * SCH, SCH)], ub[b], sub[b])

    def wait(b):
        pltpu.make_async_copy(u_hbm.at[pl.ds(base, SCH)], ub[b], sub[b]).wait()

    start(0, 0)
    start(1, 1)

    def step(j, c):
        for b in range(2):
            i = 2 * j + b
            wait(b)
            pltpu.sync_copy(ub[b], shared.at[slab_v.at[i]], add=True)

            @pl.when(i + 2 < SNCH)
            def _():
                start(i + 2, b)
        return c

    lax.fori_loop(0, SNCH // 2, step, 0)
    wait(0)
    pltpu.sync_copy(ub[0], shared.at[slab_v.at[SNCH - 1]], add=True)
    plsc.subcore_barrier()

    @pl.when(sid == 0)
    def _():
        pltpu.sync_copy(shared, s_hbm.at[cid])


def _k5(u, dst2, zeros):
    mesh = plsc.VectorSubcoreMesh(core_axis_name="c", subcore_axis_name="s")
    f = pl.kernel(
        _k5_body,
        mesh=mesh,
        out_type=jax.ShapeDtypeStruct((2, N, UW), jnp.float32),
        scratch_types=[
            pltpu.VMEM((SNCH, SCH), jnp.int32),
            pltpu.VMEM((SCH, UW), jnp.float32),
            pltpu.VMEM((SCH, UW), jnp.float32),
            pltpu.SemaphoreType.DMA,
            pltpu.SemaphoreType.DMA,
            pltpu.VMEM_SHARED((N, UW), jnp.float32),
        ],
    )
    return f(u, dst2, zeros)


BN6 = 2000


def _k6_body(s_ref, o_ref):
    den = s_ref[0, :, 0:NH] + s_ref[1, :, 0:NH]
    r = 1.0 / (den + 1e-16)
    for c in range(3):
        a = s_ref[0, :, (c + 1) * NH:(c + 2) * NH] + s_ref[1, :, (c + 1) * NH:(c + 2) * NH]
        o_ref[:, c:c + 1] = jnp.sum(a * r, axis=-1, keepdims=True) * (1.0 / NH)
    o_ref[:, 3:4] = jnp.zeros((BN6, 1), jnp.float32)


def _k6(s):
    nb = N // BN6
    return pl.pallas_call(
        _k6_body,
        grid=(nb,),
        in_specs=[pl.BlockSpec((2, BN6, UW), lambda i: (0, i, 0))],
        out_specs=pl.BlockSpec((BN6, 4), lambda i: (i, 0)),
        out_shape=jax.ShapeDtypeStruct((N, 4), jnp.float32),
    )(s)


def kernel(h, rel_x, r_feat, edge_feat, edge_index, xk_W1, xk_b1, xk_g, xk_be,
           xk_W2, xk_b2, xv_W1, xv_b1, xv_g, xv_be, xv_W2, xv_b2,
           xq_W1, xq_b1, xq_g, xq_be, xq_W2, xq_b2, ew_W, ew_b):
    src = edge_index[0]
    dst = edge_index[1]
    row = lambda x: x.reshape(1, -1)

    tdst, tsrc = _k1(
        h, xq_W1, row(xq_b1), row(xq_g), row(xq_be), xq_W2, row(xq_b2),
        xk_W1[80:336], xk_W1[336:592], xv_W1[80:336], xv_W1[336:592])

    def pack_block(b):
        b16 = lax.bitcast_convert_type(b.astype(jnp.bfloat16), jnp.uint16)
        lo = b16[:, :128].astype(jnp.uint32)
        hi = b16[:, 128:].astype(jnp.uint32)
        return lax.bitcast_convert_type(lo | (hi << 16), jnp.int32)

    tdst_p = jnp.concatenate([pack_block(tdst[:, i * D:(i + 1) * D]) for i in range(3)], 1)
    tsrc_p = jnp.concatenate([pack_block(tsrc[:, i * D:(i + 1) * D]) for i in range(2)], 1)
    gd, gs = _k2(tdst_p, tsrc_p, dst, src)

    wer_e = jnp.concatenate([xk_W1[0:16], xv_W1[0:16]], axis=1)
    wer_r = jnp.concatenate([xk_W1[16:80], xv_W1[16:80]], axis=1)
    wew = jnp.zeros((64, 8), jnp.float32).at[:, 0].set(ew_W[:, 0])
    hmat = (jax.lax.broadcasted_iota(jnp.int32, (D, NH), 0) // HD
            == jax.lax.broadcasted_iota(jnp.int32, (D, NH), 1)).astype(jnp.float32)

    sv, m = _k3(edge_feat, r_feat, gd, gs, wer_e, wer_r, wew,
                   row(xk_b1), row(xk_g), row(xk_be), xk_W2, row(xk_b2),
                   row(xv_b1), row(xv_g), row(xv_be), xv_W2, row(xv_b2),
                   hmat, ew_b.reshape(1, 1))

    u = _k4(sv, rel_x, m)

    sacc = _k5(u, dst.reshape(32, SNCH, SCH), jnp.zeros((N, UW), jnp.float32))

    out = _k6(sacc)
    return out[:, :3]


# speedup vs baseline: 19.7230x; 1.1220x over previous
import jax
import jax.numpy as jnp
from jax import lax
from jax.experimental import pallas as pl
from jax.experimental.pallas import tpu as pltpu
from jax.experimental.pallas import tpu_sc as plsc

N = 10000
E = 160000
D = 256
NH = 16
HD = 16
UW = 128


BN = 400


def _ln_relu(x, g, b):
    mu = jnp.mean(x, axis=-1, keepdims=True)
    var = jnp.mean((x - mu) ** 2, axis=-1, keepdims=True)
    x = (x - mu) * lax.rsqrt(var + 1e-5) * g + b
    return jnp.maximum(x, 0.0)


def _k1_body(h_ref, wq1_ref, bq1_ref, gq_ref, beq_ref, wq2_ref, bq2_ref,
             wk_hi_ref, wk_hj_ref, wv_hi_ref, wv_hj_ref,
             tdst_ref, tsrc_ref):
    h = h_ref[...]
    x = jnp.dot(h, wq1_ref[...], preferred_element_type=jnp.float32) + bq1_ref[...]
    x = _ln_relu(x, gq_ref[...], beq_ref[...])
    q = jnp.dot(x, wq2_ref[...], preferred_element_type=jnp.float32) + bq2_ref[...]
    tdst_ref[:, 0:D] = q
    tdst_ref[:, D:2 * D] = jnp.dot(h, wk_hi_ref[...], preferred_element_type=jnp.float32)
    tdst_ref[:, 2 * D:3 * D] = jnp.dot(h, wv_hi_ref[...], preferred_element_type=jnp.float32)
    tsrc_ref[:, 0:D] = jnp.dot(h, wk_hj_ref[...], preferred_element_type=jnp.float32)
    tsrc_ref[:, D:2 * D] = jnp.dot(h, wv_hj_ref[...], preferred_element_type=jnp.float32)


def _k1(h, wq1, bq1, gq, beq, wq2, bq2, wk_hi, wk_hj, wv_hi, wv_hj):
    nb = N // BN
    full = lambda shape: pl.BlockSpec(shape, lambda i: (0, 0))
    return pl.pallas_call(
        _k1_body,
        grid=(nb,),
        in_specs=[
            pl.BlockSpec((BN, D), lambda i: (i, 0)),
            full((D, D)), full((1, D)), full((1, D)), full((1, D)),
            full((D, D)), full((1, D)),
            full((D, D)), full((D, D)), full((D, D)), full((D, D)),
        ],
        out_specs=[
            pl.BlockSpec((BN, 3 * D), lambda i: (i, 0)),
            pl.BlockSpec((BN, 2 * D), lambda i: (i, 0)),
        ],
        out_shape=[
            jax.ShapeDtypeStruct((N, 3 * D), jnp.float32),
            jax.ShapeDtypeStruct((N, 2 * D), jnp.float32),
        ],
    )(h, wq1, bq1, gq, beq, wq2, bq2, wk_hi, wk_hj, wv_hi, wv_hj)


EPW = E // 32
CH = 40
NCH = EPW // CH
G_CH = 96
G_N = EPW // G_CH
G_TAIL = EPW - G_N * G_CH


def _k2_body(tdst_hbm, tsrc_hbm, dst_hbm, src_hbm, gd_hbm, gs_hbm,
             id0, id1, is0, is1, rd0, rd1, rs0, rs1, sd0, sd1, ss0, ss1):
    wid = lax.axis_index("s") * 2 + lax.axis_index("c")
    base = wid * EPW
    idb = (id0, id1)
    isb = (is0, is1)
    rd = (rd0, rd1)
    rs = (rs0, rs1)
    sdm = (sd0, sd1)
    ssm = (ss0, ss1)

    def start(off, n, b):
        pltpu.sync_copy(dst_hbm.at[pl.ds(off, G_CH)], idb[b])
        pltpu.sync_copy(src_hbm.at[pl.ds(off, G_CH)], isb[b])
        pltpu.async_copy(tdst_hbm.at[idb[b]], rd[b], sdm[b])
        pltpu.async_copy(tsrc_hbm.at[isb[b]], rs[b], ssm[b])

    def wait(b):
        pltpu.make_async_copy(tdst_hbm.at[idb[b]], rd[b], sdm[b]).wait()
        pltpu.make_async_copy(tsrc_hbm.at[isb[b]], rs[b], ssm[b]).wait()

    def out(off, b):
        pltpu.sync_copy(rd[b], gd_hbm.at[pl.ds(off, G_CH)])
        pltpu.sync_copy(rs[b], gs_hbm.at[pl.ds(off, G_CH)])

    start(base, G_CH, 0)
    start(base + G_CH, G_CH, 1)

    def step(j, c):
        for b in range(2):
            i = 2 * j + b
            wait(b)
            out(base + i * G_CH, b)

            @pl.when(i + 2 < G_N)
            def _():
                start(base + (i + 2) * G_CH, G_CH, b)
        return c

    lax.fori_loop(0, G_N // 2, step, 0)
    toff = base + G_N * G_CH
    pltpu.sync_copy(dst_hbm.at[pl.ds(toff, G_TAIL)], id0.at[pl.ds(0, G_TAIL)])
    pltpu.sync_copy(src_hbm.at[pl.ds(toff, G_TAIL)], is0.at[pl.ds(0, G_TAIL)])
    pltpu.async_copy(tdst_hbm.at[id0.at[pl.ds(0, G_TAIL)]], rd0.at[pl.ds(0, G_TAIL)], sd0).wait()
    pltpu.async_copy(tsrc_hbm.at[is0.at[pl.ds(0, G_TAIL)]], rs0.at[pl.ds(0, G_TAIL)], ss0).wait()
    pltpu.sync_copy(rd0.at[pl.ds(0, G_TAIL)], gd_hbm.at[pl.ds(toff, G_TAIL)])
    pltpu.sync_copy(rs0.at[pl.ds(0, G_TAIL)], gs_hbm.at[pl.ds(toff, G_TAIL)])


def _k2(tdst, tsrc, dst, src):
    mesh = plsc.VectorSubcoreMesh(core_axis_name="c", subcore_axis_name="s")
    f = pl.kernel(
        _k2_body,
        mesh=mesh,
        out_type=[
            jax.ShapeDtypeStruct((E, 384), jnp.int32),
            jax.ShapeDtypeStruct((E, 256), jnp.int32),
        ],
        scratch_types=[
            pltpu.VMEM((G_CH,), jnp.int32),
            pltpu.VMEM((G_CH,), jnp.int32),
            pltpu.VMEM((G_CH,), jnp.int32),
            pltpu.VMEM((G_CH,), jnp.int32),
            pltpu.VMEM((G_CH, 384), jnp.int32),
            pltpu.VMEM((G_CH, 384), jnp.int32),
            pltpu.VMEM((G_CH, 256), jnp.int32),
            pltpu.VMEM((G_CH, 256), jnp.int32),
            pltpu.SemaphoreType.DMA,
            pltpu.SemaphoreType.DMA,
            pltpu.SemaphoreType.DMA,
            pltpu.SemaphoreType.DMA,
        ],
    )
    return f(tdst, tsrc, dst, src)


BE = 640


def _unpack(xi32):
    lo = lax.bitcast_convert_type(xi32 << 16, jnp.float32)
    hi = lax.bitcast_convert_type(xi32 & jnp.int32(-65536), jnp.float32)
    return jnp.concatenate([lo, hi], axis=1)


def _k3_body(ef_ref, rf_ref, gd_ref, gs_ref, wer_e_ref, wer_r_ref, wew_ref,
             bk1_ref, gk_ref, bek_ref, wk2_ref, bk2_ref,
             bv1_ref, gv_ref, bev_ref, wv2_ref, bv2_ref,
             hmat_ref, ewb_ref, rx_ref,
             u_ref):
    ef = ef_ref[...]
    rf = rf_ref[...]
    pk_d = _unpack(gd_ref[:, 128:256])
    pv_d = _unpack(gd_ref[:, 256:384])
    pk_s = _unpack(gs_ref[:, 0:128])
    pv_s = _unpack(gs_ref[:, 128:256])
    b16 = lambda x: x.astype(jnp.bfloat16)
    pre = (jnp.dot(b16(ef), b16(wer_e_ref[...]), preferred_element_type=jnp.float32)
           + jnp.dot(b16(rf), b16(wer_r_ref[...]), preferred_element_type=jnp.float32))
    pre_k = pre[:, 0:D] + pk_d + pk_s + bk1_ref[...]
    pre_v = pre[:, D:2 * D] + pv_d + pv_s + bv1_ref[...]
    xk = _ln_relu(pre_k, gk_ref[...], bek_ref[...])
    xv = _ln_relu(pre_v, gv_ref[...], bev_ref[...])
    k = jnp.dot(b16(xk), b16(wk2_ref[...]), preferred_element_type=jnp.float32) + bk2_ref[...]
    v = jnp.dot(b16(xv), b16(wv2_ref[...]), preferred_element_type=jnp.float32) + bv2_ref[...]
    qd = _unpack(gd_ref[:, 0:128])
    s = jnp.dot(b16(qd * k), b16(hmat_ref[...]), preferred_element_type=jnp.float32) * 0.25
    ew_pre = jnp.dot(rf, wew_ref[...], preferred_element_type=jnp.float32)
    e_w = jax.nn.sigmoid(ew_pre[:, 0:1] + ewb_ref[0, 0])
    t = jnp.exp(jnp.clip(s, -60.0, 60.0))
    p = t * (v * e_w)
    rx = rx_ref[...]
    u_ref[:, 0:NH] = t
    u_ref[:, NH:2 * NH] = p * rx[:, 0:1]
    u_ref[:, 2 * NH:3 * NH] = p * rx[:, 1:2]
    u_ref[:, 3 * NH:4 * NH] = p * rx[:, 2:3]
    u_ref[:, 4 * NH:UW] = jnp.zeros((BE, UW - 4 * NH), jnp.float32)


def _k3(ef, rf, gd, gs, wer_e, wer_r, wew, bk1, gk, bek, wk2, bk2, bv1, gv, bev, wv2, bv2, hmat, ewb, rel_x):
    nb = E // BE
    full = lambda shape: pl.BlockSpec(shape, lambda i: (0, 0))
    return pl.pallas_call(
        _k3_body,
        grid=(nb,),
        in_specs=[
            pl.BlockSpec((BE, 16), lambda i: (i, 0)),
            pl.BlockSpec((BE, 64), lambda i: (i, 0)),
            pl.BlockSpec((BE, 384), lambda i: (i, 0)),
            pl.BlockSpec((BE, 256), lambda i: (i, 0)),
            full((16, 2 * D)), full((64, 2 * D)), full((64, 8)),
            full((1, D)), full((1, D)), full((1, D)), full((D, D)), full((1, D)),
            full((1, D)), full((1, D)), full((1, D)), full((D, NH)), full((1, NH)),
            full((D, NH)), full((1, 1)),
            pl.BlockSpec((BE, 3), lambda i: (i, 0)),
        ],
        out_specs=pl.BlockSpec((BE, UW), lambda i: (i, 0)),
        out_shape=jax.ShapeDtypeStruct((E, UW), jnp.float32),
    )(ef, rf, gd, gs, wer_e, wer_r, wew, bk1, gk, bek, wk2, bk2, bv1, gv, bev, wv2, bv2, hmat, ewb, rel_x)


SCH = 40
SNCH = EPW // SCH


def _k5_body(u_hbm, dst2_hbm, zeros_hbm, s_hbm, slab_v, u0, u1, su0, su1, shared):
    cid = lax.axis_index("c")
    sid = lax.axis_index("s")
    wid = cid * 16 + sid
    base = wid * EPW

    @pl.when(sid == 0)
    def _():
        pltpu.sync_copy(zeros_hbm, shared)

    pltpu.sync_copy(dst2_hbm.at[wid], slab_v)
    plsc.subcore_barrier()

    ub = (u0, u1)
    sub = (su0, su1)

    def start(i, b):
        pltpu.async_copy(u_hbm.at[pl.ds(base + i * SCH, SCH)], ub[b], sub[b])

    def wait(b):
        pltpu.make_async_copy(u_hbm.at[pl.ds(base, SCH)], ub[b], sub[b]).wait()

    start(0, 0)
    start(1, 1)

    def step(j, c):
        for b in range(2):
            i = 2 * j + b
            wait(b)
            pltpu.sync_copy(ub[b], shared.at[slab_v.at[i]], add=True)

            @pl.when(i + 2 < SNCH)
            def _():
                start(i + 2, b)
        return c

    lax.fori_loop(0, SNCH // 2, step, 0)
    wait(0)
    pltpu.sync_copy(ub[0], shared.at[slab_v.at[SNCH - 1]], add=True)
    plsc.subcore_barrier()

    @pl.when(sid == 0)
    def _():
        pltpu.sync_copy(shared, s_hbm.at[cid])


def _k5(u, dst2, zeros):
    mesh = plsc.VectorSubcoreMesh(core_axis_name="c", subcore_axis_name="s")
    f = pl.kernel(
        _k5_body,
        mesh=mesh,
        out_type=jax.ShapeDtypeStruct((2, N, UW), jnp.float32),
        scratch_types=[
            pltpu.VMEM((SNCH, SCH), jnp.int32),
            pltpu.VMEM((SCH, UW), jnp.float32),
            pltpu.VMEM((SCH, UW), jnp.float32),
            pltpu.SemaphoreType.DMA,
            pltpu.SemaphoreType.DMA,
            pltpu.VMEM_SHARED((N, UW), jnp.float32),
        ],
    )
    return f(u, dst2, zeros)


BN6 = 2000


def _k6_body(s_ref, o_ref):
    den = s_ref[0, :, 0:NH] + s_ref[1, :, 0:NH]
    r = 1.0 / (den + 1e-16)
    for c in range(3):
        a = s_ref[0, :, (c + 1) * NH:(c + 2) * NH] + s_ref[1, :, (c + 1) * NH:(c + 2) * NH]
        o_ref[:, c:c + 1] = jnp.sum(a * r, axis=-1, keepdims=True) * (1.0 / NH)
    o_ref[:, 3:4] = jnp.zeros((BN6, 1), jnp.float32)


def _k6(s):
    nb = N // BN6
    return pl.pallas_call(
        _k6_body,
        grid=(nb,),
        in_specs=[pl.BlockSpec((2, BN6, UW), lambda i: (0, i, 0))],
        out_specs=pl.BlockSpec((BN6, 4), lambda i: (i, 0)),
        out_shape=jax.ShapeDtypeStruct((N, 4), jnp.float32),
    )(s)


def kernel(h, rel_x, r_feat, edge_feat, edge_index, xk_W1, xk_b1, xk_g, xk_be,
           xk_W2, xk_b2, xv_W1, xv_b1, xv_g, xv_be, xv_W2, xv_b2,
           xq_W1, xq_b1, xq_g, xq_be, xq_W2, xq_b2, ew_W, ew_b):
    src = edge_index[0]
    dst = edge_index[1]
    row = lambda x: x.reshape(1, -1)

    tdst, tsrc = _k1(
        h, xq_W1, row(xq_b1), row(xq_g), row(xq_be), xq_W2, row(xq_b2),
        xk_W1[80:336], xk_W1[336:592], xv_W1[80:336], xv_W1[336:592])

    def pack_block(b):
        b16 = lax.bitcast_convert_type(b.astype(jnp.bfloat16), jnp.uint16)
        lo = b16[:, :128].astype(jnp.uint32)
        hi = b16[:, 128:].astype(jnp.uint32)
        return lax.bitcast_convert_type(lo | (hi << 16), jnp.int32)

    tdst_p = jnp.concatenate([pack_block(tdst[:, i * D:(i + 1) * D]) for i in range(3)], 1)
    tsrc_p = jnp.concatenate([pack_block(tsrc[:, i * D:(i + 1) * D]) for i in range(2)], 1)
    gd, gs = _k2(tdst_p, tsrc_p, dst, src)

    wer_e = jnp.concatenate([xk_W1[0:16], xv_W1[0:16]], axis=1)
    wer_r = jnp.concatenate([xk_W1[16:80], xv_W1[16:80]], axis=1)
    wew = jnp.zeros((64, 8), jnp.float32).at[:, 0].set(ew_W[:, 0])
    hmat = (jax.lax.broadcasted_iota(jnp.int32, (D, NH), 0) // HD
            == jax.lax.broadcasted_iota(jnp.int32, (D, NH), 1)).astype(jnp.float32)

    u = _k3(edge_feat, r_feat, gd, gs, wer_e, wer_r, wew,
            row(xk_b1), row(xk_g), row(xk_be), xk_W2, row(xk_b2),
            row(xv_b1), row(xv_g), row(xv_be), xv_W2, row(xv_b2),
            hmat, ew_b.reshape(1, 1), rel_x)

    sacc = _k5(u, dst.reshape(32, SNCH, SCH), jnp.zeros((N, UW), jnp.float32))

    out = _k6(sacc)
    return out[:, :3]
